# Initial kernel scaffold; baseline (speedup 1.0000x reference)
import jax
import jax.numpy as jnp
from jax.experimental import pallas as pl


def kernel(x, edge_index, batch, emb, W0, b0, W1, b1, W2, b2, W_ih, W_hh, b_ih, b_hh, lin0_W, lin0_b, lin3_W, lin3_b):
    raise NotImplementedError("write your pallas kernel here")


# speedup vs baseline: 12.2823x; 12.2823x over previous
import functools

import jax
import jax.numpy as jnp
from jax import lax
from jax.experimental import pallas as pl
from jax.experimental.pallas import tpu as pltpu
from jax.experimental.pallas import tpu_sc as plsc

F32 = jnp.float32
I32 = jnp.int32

NCORES = 2
NSUB = 16
ROW = 128
BLK = 1024
NSEG = 256

_SC_PARAMS = pltpu.CompilerParams(use_tc_tiling_on_sc=False)


def _ceil_to(a, m):
    return (a + m - 1) // m * m


def _make_deg_kernel(e_pad, n_pad):
    rows_w = e_pad // ROW // (NCORES * NSUB)
    n_slice = n_pad // NSUB
    mesh = plsc.VectorSubcoreMesh(core_axis_name="c", subcore_axis_name="s")

    @functools.partial(
        pl.kernel,
        mesh=mesh,
        compiler_params=_SC_PARAMS,
        out_type=jax.ShapeDtypeStruct((NCORES * n_pad,), F32),
        scratch_types=[
            pltpu.VMEM((rows_w, ROW), I32),
            pltpu.VMEM((n_slice,), F32),
            pltpu.VMEM((ROW,), F32),
            pltpu.VMEM_SHARED((n_pad,), F32),
        ],
    )
    def deg_kernel(dst_hbm, out_hbm, idx_v, zero_v, ones_v, acc_sh):
        cid = lax.axis_index("c")
        sid = lax.axis_index("s")

        @pl.loop(0, n_slice // 16)
        def _(i):
            zero_v[pl.ds(i * 16, 16)] = jnp.zeros((16,), F32)

        @pl.loop(0, ROW // 16)
        def _(i):
            ones_v[pl.ds(i * 16, 16)] = jnp.ones((16,), F32)

        pltpu.sync_copy(zero_v, acc_sh.at[pl.ds(sid * n_slice, n_slice)])
        plsc.subcore_barrier()

        wid = cid * NSUB + sid
        pltpu.sync_copy(dst_hbm.at[pl.ds(wid * rows_w, rows_w)], idx_v)

        @pl.loop(0, rows_w)
        def _(j):
            pltpu.sync_copy(ones_v, acc_sh.at[idx_v.at[j]], add=True)

        plsc.subcore_barrier()
        out_off = cid * n_pad + sid * n_slice
        pltpu.sync_copy(acc_sh.at[pl.ds(sid * n_slice, n_slice)],
                        out_hbm.at[pl.ds(out_off, n_slice)])

    return deg_kernel


def _make_layer_kernel(e_pad, n_pad):
    rows_w = e_pad // ROW // NSUB
    ch_rows = 8
    n_chunks = rows_w // ch_rows
    ch_e = ch_rows * ROW
    n_slice = n_pad // NSUB
    mesh = plsc.VectorSubcoreMesh(core_axis_name="c", subcore_axis_name="s")

    @functools.partial(
        pl.kernel,
        mesh=mesh,
        compiler_params=_SC_PARAMS,
        out_type=jax.ShapeDtypeStruct((NCORES * n_pad, 16), F32),
        scratch_types=[
            pltpu.VMEM((ch_rows, ROW), I32),
            pltpu.VMEM((ch_rows, ROW), I32),
            pltpu.VMEM((ch_e, 16), F32),
            pltpu.VMEM_SHARED((n_pad, 16), F32),
        ],
    )
    def layer_kernel(hw0_hbm, hw1_hbm, src_hbm, dst_hbm, out_hbm,
                     src_v, dst_v, msg_v, acc_sh):
        cid = lax.axis_index("c")
        sid = lax.axis_index("s")

        @pl.loop(0, ch_e)
        def _(i):
            msg_v[i] = jnp.zeros((16,), F32)

        base_n = sid * n_slice

        @pl.loop(0, n_slice // ch_e)
        def _(i):
            pltpu.sync_copy(msg_v, acc_sh.at[pl.ds(base_n + i * ch_e, ch_e)])

        rem = n_slice % ch_e
        if rem:
            pltpu.sync_copy(msg_v.at[pl.ds(0, rem)],
                            acc_sh.at[pl.ds(base_n + (n_slice // ch_e) * ch_e,
                                            rem)])
        plsc.subcore_barrier()

        row_base = sid * rows_w

        def gather_from(hw_ref):
            @pl.loop(0, ch_rows)
            def _(j):
                pltpu.sync_copy(hw_ref.at[src_v.at[j]],
                                msg_v.at[pl.ds(j * ROW, ROW)])

        @pl.loop(0, n_chunks)
        def _(g):
            rb = row_base + g * ch_rows
            pltpu.sync_copy(src_hbm.at[pl.ds(rb, ch_rows)], src_v)
            pltpu.sync_copy(dst_hbm.at[pl.ds(rb, ch_rows)], dst_v)

            @pl.when(cid == 0)
            def _():
                gather_from(hw0_hbm)

            @pl.when(cid == 1)
            def _():
                gather_from(hw1_hbm)

            @pl.loop(0, ch_rows)
            def _(j):
                pltpu.sync_copy(msg_v.at[pl.ds(j * ROW, ROW)],
                                acc_sh.at[dst_v.at[j]], add=True)

        plsc.subcore_barrier()
        out_off = cid * n_pad + base_n
        pltpu.sync_copy(acc_sh.at[pl.ds(base_n, n_slice)],
                        out_hbm.at[pl.ds(out_off, n_slice)])

    return layer_kernel


def _tc_prep(deg4, x3, embp, w0, n_pad):
    nb = n_pad // BLK

    def body(deg_ref, x_ref, emb_ref, w0_ref, hw_ref, dis_ref):
        deg = deg_ref[0, 0] + deg_ref[1, 0] + 1.0
        dis = lax.rsqrt(deg)
        dis_ref[0] = dis
        xv = x_ref[0]
        cls = lax.broadcasted_iota(I32, (BLK, 128), 1)
        oh = (xv == cls).astype(F32)
        h0 = jnp.dot(oh, emb_ref[...], preferred_element_type=F32)
        th = dis * h0
        hw1 = jnp.dot(th, w0_ref[...], preferred_element_type=F32)
        hw_ref[0] = hw1[:, :16]
        hw_ref[1] = hw1[:, 16:]

    return pl.pallas_call(
        body,
        grid=(nb,),
        in_specs=[
            pl.BlockSpec((2, 1, BLK, 1), lambda i: (0, i, 0, 0)),
            pl.BlockSpec((1, BLK, 1), lambda i: (i, 0, 0)),
            pl.BlockSpec((128, 32), lambda i: (0, 0)),
            pl.BlockSpec((32, 32), lambda i: (0, 0)),
        ],
        out_specs=[
            pl.BlockSpec((2, BLK, 16), lambda i: (0, i, 0)),
            pl.BlockSpec((1, BLK, 1), lambda i: (i, 0, 0)),
        ],
        out_shape=[
            jax.ShapeDtypeStruct((2, n_pad, 16), F32),
            jax.ShapeDtypeStruct((nb, BLK, 1), F32),
        ],
    )(deg4, x3, embp, w0)


def _tc_layer(acc, hwp, dis3, bvec, wnext, n_pad):
    nb = n_pad // BLK

    def body(acc_ref, hw_ref, dis_ref, b_ref, w_ref, out_ref):
        s = jnp.concatenate(
            [acc_ref[0] + hw_ref[0], acc_ref[1] + hw_ref[1]], axis=1)
        d = dis_ref[0]
        h = jnp.maximum(d * s + b_ref[...], 0.0)
        o = jnp.dot(d * h, w_ref[...], preferred_element_type=F32)
        out_ref[0] = o[:, :16]
        out_ref[1] = o[:, 16:]

    return pl.pallas_call(
        body,
        grid=(nb,),
        in_specs=[
            pl.BlockSpec((2, BLK, 16), lambda i: (0, i, 0)),
            pl.BlockSpec((2, BLK, 16), lambda i: (0, i, 0)),
            pl.BlockSpec((1, BLK, 1), lambda i: (i, 0, 0)),
            pl.BlockSpec((1, 32), lambda i: (0, 0)),
            pl.BlockSpec((32, 32), lambda i: (0, 0)),
        ],
        out_specs=pl.BlockSpec((2, BLK, 16), lambda i: (0, i, 0)),
        out_shape=jax.ShapeDtypeStruct((2, n_pad, 16), F32),
    )(acc, hwp, dis3, bvec, wnext)


def _lstm_step1(b_ih_ref, b_hh_ref):
    gb = b_ih_ref[...] + b_hh_ref[...]
    ii = jax.nn.sigmoid(gb[:, 0:32])
    ff = jax.nn.sigmoid(gb[:, 32:64])
    gg = jnp.tanh(gb[:, 64:96])
    oo = jax.nn.sigmoid(gb[:, 96:128])
    c1 = ii * gg
    h1 = oo * jnp.tanh(c1)
    del ff
    return c1, h1


def _block_h3(acc_ref, hw_ref, dis_ref, b2_ref):
    s = jnp.concatenate(
        [acc_ref[0] + hw_ref[0], acc_ref[1] + hw_ref[1]], axis=1)
    d = dis_ref[0]
    return jnp.maximum(d * s + b2_ref[...], 0.0)


def _block_onehot(batch_ref):
    bv = batch_ref[0]
    seg = lax.broadcasted_iota(I32, (BLK, NSEG), 1)
    return bv == seg


def _seg_sweep_a(h3, oh, q):
    hq = lax.dot_general(h3, q, (((1,), (1,)), ((), ())),
                         preferred_element_type=F32)
    masked = jnp.where(oh, hq, -jnp.inf)
    return jnp.max(masked, axis=0, keepdims=True)


def _seg_sweep_b(h3, oh, q, emax):
    ohf = oh.astype(F32)
    hq = lax.dot_general(h3, q, (((1,), (1,)), ((), ())),
                         preferred_element_type=F32)
    e = jnp.sum(ohf * hq, axis=1, keepdims=True)
    emaxf = jnp.where(jnp.isfinite(emax), emax, 0.0)
    emv = lax.dot_general(ohf, emaxf, (((1,), (1,)), ((), ())),
                          preferred_element_type=F32)
    ee = jnp.exp(e - emv) * jnp.sum(ohf, axis=1, keepdims=True)
    esum = jnp.sum(ohf * ee, axis=0, keepdims=True)
    rnum = lax.dot_general(ohf, ee * h3, (((0,), (0,)), ((), ())),
                           preferred_element_type=F32)
    return esum, rnum


_S2S_SPECS = [
    pl.BlockSpec((2, BLK, 16), lambda i: (0, i, 0)),
    pl.BlockSpec((2, BLK, 16), lambda i: (0, i, 0)),
    pl.BlockSpec((1, BLK, 1), lambda i: (i, 0, 0)),
    pl.BlockSpec((1, BLK, 1), lambda i: (i, 0, 0)),
    pl.BlockSpec((1, 32), lambda i: (0, 0)),
]


def _tc_s2s_a(acc, hwp, dis3, batch3, b2, b_ih, b_hh, n_pad):
    nb = n_pad // BLK

    def body(acc_ref, hw_ref, dis_ref, bat_ref, b2_ref, bih_ref, bhh_ref,
             emax_ref):
        i = pl.program_id(0)

        @pl.when(i == 0)
        def _():
            emax_ref[...] = jnp.full((1, NSEG), -jnp.inf, F32)

        _, h1 = _lstm_step1(bih_ref, bhh_ref)
        q1 = jnp.broadcast_to(h1, (NSEG, 32))
        h3 = _block_h3(acc_ref, hw_ref, dis_ref, b2_ref)
        oh = _block_onehot(bat_ref)
        emax_ref[...] = jnp.maximum(emax_ref[...], _seg_sweep_a(h3, oh, q1))

    return pl.pallas_call(
        body,
        grid=(nb,),
        in_specs=_S2S_SPECS + [
            pl.BlockSpec((1, 128), lambda i: (0, 0)),
            pl.BlockSpec((1, 128), lambda i: (0, 0)),
        ],
        out_specs=pl.BlockSpec((1, NSEG), lambda i: (0, 0)),
        out_shape=jax.ShapeDtypeStruct((1, NSEG), F32),
    )(acc, hwp, dis3, batch3, b2, b_ih, b_hh)


def _tc_s2s_b(acc, hwp, dis3, batch3, b2, b_ih, b_hh, emax1, n_pad):
    nb = n_pad // BLK

    def body(acc_ref, hw_ref, dis_ref, bat_ref, b2_ref, bih_ref, bhh_ref,
             emax_ref, esum_ref, rnum_ref):
        i = pl.program_id(0)

        @pl.when(i == 0)
        def _():
            esum_ref[...] = jnp.zeros((1, NSEG), F32)
            rnum_ref[...] = jnp.zeros((NSEG, 32), F32)

        _, h1 = _lstm_step1(bih_ref, bhh_ref)
        q1 = jnp.broadcast_to(h1, (NSEG, 32))
        h3 = _block_h3(acc_ref, hw_ref, dis_ref, b2_ref)
        oh = _block_onehot(bat_ref)
        esum, rnum = _seg_sweep_b(h3, oh, q1, emax_ref[...])
        esum_ref[...] += esum
        rnum_ref[...] += rnum

    return pl.pallas_call(
        body,
        grid=(nb,),
        in_specs=_S2S_SPECS + [
            pl.BlockSpec((1, 128), lambda i: (0, 0)),
            pl.BlockSpec((1, 128), lambda i: (0, 0)),
            pl.BlockSpec((1, NSEG), lambda i: (0, 0)),
        ],
        out_specs=[
            pl.BlockSpec((1, NSEG), lambda i: (0, 0)),
            pl.BlockSpec((NSEG, 32), lambda i: (0, 0)),
        ],
        out_shape=[
            jax.ShapeDtypeStruct((1, NSEG), F32),
            jax.ShapeDtypeStruct((NSEG, 32), F32),
        ],
    )(acc, hwp, dis3, batch3, b2, b_ih, b_hh, emax1)


def _lstm_step2(bih_ref, bhh_ref, wih_ref, whh_ref, esum1, rnum1):
    c1, h1 = _lstm_step1(bih_ref, bhh_ref)
    q1 = jnp.broadcast_to(h1, (NSEG, 32))
    r1 = rnum1 / (jnp.transpose(esum1) + 1e-16)
    qs1 = jnp.concatenate([q1, r1], axis=1)
    gates = (jnp.dot(qs1, wih_ref[...], preferred_element_type=F32)
             + bih_ref[...]
             + jnp.dot(jnp.broadcast_to(h1, (NSEG, 32)), whh_ref[...],
                       preferred_element_type=F32)
             + bhh_ref[...])
    i2 = jax.nn.sigmoid(gates[:, 0:32])
    f2 = jax.nn.sigmoid(gates[:, 32:64])
    g2 = jnp.tanh(gates[:, 64:96])
    o2 = jax.nn.sigmoid(gates[:, 96:128])
    c2 = f2 * c1 + i2 * g2
    h2 = o2 * jnp.tanh(c2)
    return h2


def _tc_s2s_c(acc, hwp, dis3, batch3, b2, b_ih, b_hh, wihT, whhT,
              esum1, rnum1, n_pad):
    nb = n_pad // BLK

    def body(acc_ref, hw_ref, dis_ref, bat_ref, b2_ref, bih_ref, bhh_ref,
             wih_ref, whh_ref, esum_ref, rnum_ref, emax_ref, q2_ref):
        i = pl.program_id(0)

        @pl.when(i == 0)
        def _():
            emax_ref[...] = jnp.full((1, NSEG), -jnp.inf, F32)

        q2 = _lstm_step2(bih_ref, bhh_ref, wih_ref, whh_ref,
                         esum_ref[...], rnum_ref[...])
        q2_ref[...] = q2
        h3 = _block_h3(acc_ref, hw_ref, dis_ref, b2_ref)
        oh = _block_onehot(bat_ref)
        emax_ref[...] = jnp.maximum(emax_ref[...], _seg_sweep_a(h3, oh, q2))

    return pl.pallas_call(
        body,
        grid=(nb,),
        in_specs=_S2S_SPECS + [
            pl.BlockSpec((1, 128), lambda i: (0, 0)),
            pl.BlockSpec((1, 128), lambda i: (0, 0)),
            pl.BlockSpec((64, 128), lambda i: (0, 0)),
            pl.BlockSpec((32, 128), lambda i: (0, 0)),
            pl.BlockSpec((1, NSEG), lambda i: (0, 0)),
            pl.BlockSpec((NSEG, 32), lambda i: (0, 0)),
        ],
        out_specs=[
            pl.BlockSpec((1, NSEG), lambda i: (0, 0)),
            pl.BlockSpec((NSEG, 32), lambda i: (0, 0)),
        ],
        out_shape=[
            jax.ShapeDtypeStruct((1, NSEG), F32),
            jax.ShapeDtypeStruct((NSEG, 32), F32),
        ],
    )(acc, hwp, dis3, batch3, b2, b_ih, b_hh, wihT, whhT, esum1, rnum1)


def _tc_s2s_d(acc, hwp, dis3, batch3, b2, emax2, q2, lin0WT, lin0b,
              lin3WT, lin3b, n_pad):
    nb = n_pad // BLK

    def body(acc_ref, hw_ref, dis_ref, bat_ref, b2_ref, emax_ref, q2_ref,
             l0w_ref, l0b_ref, l3w_ref, l3b_ref, out_ref,
             esum_s, rnum_s):
        i = pl.program_id(0)

        @pl.when(i == 0)
        def _():
            esum_s[...] = jnp.zeros((1, NSEG), F32)
            rnum_s[...] = jnp.zeros((NSEG, 32), F32)

        q2 = q2_ref[...]
        h3 = _block_h3(acc_ref, hw_ref, dis_ref, b2_ref)
        oh = _block_onehot(bat_ref)
        esum, rnum = _seg_sweep_b(h3, oh, q2, emax_ref[...])
        esum_s[...] += esum
        rnum_s[...] += rnum

        @pl.when(i == nb - 1)
        def _():
            r2 = rnum_s[...] / (jnp.transpose(esum_s[...]) + 1e-16)
            qs2 = jnp.concatenate([q2, r2], axis=1)
            z = jnp.maximum(
                jnp.dot(qs2, l0w_ref[...], preferred_element_type=F32)
                + l0b_ref[...], 0.0)
            out_ref[...] = (jnp.dot(z, l3w_ref[...],
                                    preferred_element_type=F32)
                            + l3b_ref[...])

    return pl.pallas_call(
        body,
        grid=(nb,),
        in_specs=_S2S_SPECS + [
            pl.BlockSpec((1, NSEG), lambda i: (0, 0)),
            pl.BlockSpec((NSEG, 32), lambda i: (0, 0)),
            pl.BlockSpec((64, 32), lambda i: (0, 0)),
            pl.BlockSpec((1, 32), lambda i: (0, 0)),
            pl.BlockSpec((32, 1), lambda i: (0, 0)),
            pl.BlockSpec((1, 1), lambda i: (0, 0)),
        ],
        out_specs=pl.BlockSpec((NSEG, 1), lambda i: (0, 0)),
        out_shape=jax.ShapeDtypeStruct((NSEG, 1), F32),
        scratch_shapes=[
            pltpu.VMEM((1, NSEG), F32),
            pltpu.VMEM((NSEG, 32), F32),
        ],
    )(acc, hwp, dis3, batch3, b2, emax2, q2, lin0WT, lin0b, lin3WT, lin3b)


def kernel(x, edge_index, batch, emb, W0, b0, W1, b1, W2, b2,
           W_ih, W_hh, b_ih, b_hh, lin0_W, lin0_b, lin3_W, lin3_b):
    n = x.shape[0]
    e = edge_index.shape[1]

    n_pad = _ceil_to(n, NSUB * ROW)
    if n_pad == n:
        n_pad += NSUB * ROW
    pad_rows = n_pad - n
    e_pad = _ceil_to(e, NCORES * NSUB * 16 * ROW)
    nb = n_pad // BLK

    src = edge_index[0].astype(I32)
    dst = edge_index[1].astype(I32)
    pe = e_pad - e
    pad_ar = jnp.arange(pe, dtype=I32)
    src_p = jnp.concatenate([src, (pad_ar * 97) % n])
    dst_p = jnp.concatenate([dst, n + pad_ar % pad_rows])
    src2d = src_p.reshape(e_pad // ROW, ROW)
    dst2d = dst_p.reshape(e_pad // ROW, ROW)

    x_p = jnp.concatenate([x.astype(I32), jnp.zeros((n_pad - n,), I32)])
    x3 = x_p.reshape(nb, BLK, 1)
    batch_p = jnp.concatenate(
        [batch.astype(I32), jnp.full((n_pad - n,), NSEG, I32)])
    batch3 = batch_p.reshape(nb, BLK, 1)

    embp = jnp.zeros((128, 32), F32).at[:emb.shape[0]].set(emb)
    b0r = b0.reshape(1, 32)
    b1r = b1.reshape(1, 32)
    b2r = b2.reshape(1, 32)
    bihr = b_ih.reshape(1, 128)
    bhhr = b_hh.reshape(1, 128)
    wihT = W_ih.T
    whhT = W_hh.T
    lin0WT = lin0_W.T
    lin0br = lin0_b.reshape(1, 32)
    lin3WT = lin3_W.T
    lin3br = lin3_b.reshape(1, 1)

    deg_flat = _make_deg_kernel(e_pad, n_pad)(dst2d)
    deg4 = deg_flat.reshape(2, nb, BLK, 1)

    hw1, dis3 = _tc_prep(deg4, x3, embp, W0, n_pad)

    layer = _make_layer_kernel(e_pad, n_pad)

    acc1 = layer(hw1[0], hw1[1], src2d, dst2d).reshape(2, n_pad, 16)
    hw2 = _tc_layer(acc1, hw1, dis3, b0r, W1, n_pad)
    acc2 = layer(hw2[0], hw2[1], src2d, dst2d).reshape(2, n_pad, 16)
    hw3 = _tc_layer(acc2, hw2, dis3, b1r, W2, n_pad)
    acc3 = layer(hw3[0], hw3[1], src2d, dst2d).reshape(2, n_pad, 16)

    emax1 = _tc_s2s_a(acc3, hw3, dis3, batch3, b2r, bihr, bhhr, n_pad)
    esum1, rnum1 = _tc_s2s_b(acc3, hw3, dis3, batch3, b2r, bihr, bhhr,
                             emax1, n_pad)
    emax2, q2 = _tc_s2s_c(acc3, hw3, dis3, batch3, b2r, bihr, bhhr,
                          wihT, whhT, esum1, rnum1, n_pad)
    out = _tc_s2s_d(acc3, hw3, dis3, batch3, b2r, emax2, q2,
                    lin0WT, lin0br, lin3WT, lin3br, n_pad)
    return out


# speedup vs baseline: 17.7862x; 1.4481x over previous
import functools

import jax
import jax.numpy as jnp
from jax import lax
from jax.experimental import pallas as pl
from jax.experimental.pallas import tpu as pltpu
from jax.experimental.pallas import tpu_sc as plsc

F32 = jnp.float32
I32 = jnp.int32

NCORES = 2
NSUB = 16
ROW = 128
BLK = 1024
NSEG = 256

_SC_PARAMS = pltpu.CompilerParams(use_tc_tiling_on_sc=False)


def _ceil_to(a, m):
    return (a + m - 1) // m * m


def _make_deg_kernel(e_pad, n_pad):
    rows_w = e_pad // ROW // (NCORES * NSUB)
    n_slice = n_pad // NSUB
    mesh = plsc.VectorSubcoreMesh(core_axis_name="c", subcore_axis_name="s")

    @functools.partial(
        pl.kernel,
        mesh=mesh,
        compiler_params=_SC_PARAMS,
        out_type=jax.ShapeDtypeStruct((NCORES * n_pad,), F32),
        scratch_types=[
            pltpu.VMEM((rows_w, ROW), I32),
            pltpu.VMEM((n_slice,), F32),
            pltpu.VMEM((ROW,), F32),
            pltpu.VMEM_SHARED((n_pad,), F32),
        ],
    )
    def deg_kernel(dst_hbm, out_hbm, idx_v, zero_v, ones_v, acc_sh):
        cid = lax.axis_index("c")
        sid = lax.axis_index("s")

        @pl.loop(0, n_slice // 16)
        def _(i):
            zero_v[pl.ds(i * 16, 16)] = jnp.zeros((16,), F32)

        @pl.loop(0, ROW // 16)
        def _(i):
            ones_v[pl.ds(i * 16, 16)] = jnp.ones((16,), F32)

        pltpu.sync_copy(zero_v, acc_sh.at[pl.ds(sid * n_slice, n_slice)])
        plsc.subcore_barrier()

        wid = cid * NSUB + sid
        pltpu.sync_copy(dst_hbm.at[pl.ds(wid * rows_w, rows_w)], idx_v)

        @pl.loop(0, rows_w)
        def _(j):
            pltpu.sync_copy(ones_v, acc_sh.at[idx_v.at[j]], add=True)

        plsc.subcore_barrier()
        out_off = cid * n_pad + sid * n_slice
        pltpu.sync_copy(acc_sh.at[pl.ds(sid * n_slice, n_slice)],
                        out_hbm.at[pl.ds(out_off, n_slice)])

    return deg_kernel


def _make_layer_kernel(e_pad, n_pad):
    rows_w = e_pad // ROW // NSUB
    ch_rows = 4
    n_chunks = rows_w // ch_rows
    ch_e = ch_rows * ROW
    n_slice = n_pad // NSUB
    mesh = plsc.VectorSubcoreMesh(core_axis_name="c", subcore_axis_name="s")

    @functools.partial(
        pl.kernel,
        mesh=mesh,
        compiler_params=_SC_PARAMS,
        out_type=jax.ShapeDtypeStruct((NCORES * n_pad, 16), F32),
        scratch_types=[
            pltpu.VMEM((ch_rows, ROW), I32),
            pltpu.VMEM((ch_rows, ROW), I32),
            pltpu.VMEM((ch_rows, ROW), I32),
            pltpu.VMEM((ch_rows, ROW), I32),
            pltpu.VMEM((ch_e, 16), F32),
            pltpu.VMEM((ch_e, 16), F32),
            pltpu.SemaphoreType.DMA,
            pltpu.SemaphoreType.DMA,
            pltpu.SemaphoreType.DMA,
            pltpu.SemaphoreType.DMA,
            pltpu.VMEM_SHARED((n_pad, 16), F32),
        ],
    )
    def layer_kernel(hw0_hbm, hw1_hbm, src_hbm, dst_hbm, out_hbm,
                     src0_v, dst0_v, src1_v, dst1_v, msg0_v, msg1_v,
                     gsem0, gsem1, ssem0, ssem1, acc_sh):
        cid = lax.axis_index("c")
        sid = lax.axis_index("s")

        @pl.loop(0, ch_e)
        def _(i):
            msg0_v[i] = jnp.zeros((16,), F32)

        base_n = sid * n_slice

        @pl.loop(0, n_slice // ch_e)
        def _(i):
            pltpu.sync_copy(msg0_v, acc_sh.at[pl.ds(base_n + i * ch_e, ch_e)])

        rem = n_slice % ch_e
        if rem:
            pltpu.sync_copy(msg0_v.at[pl.ds(0, rem)],
                            acc_sh.at[pl.ds(base_n + (n_slice // ch_e) * ch_e,
                                            rem)])
        plsc.subcore_barrier()

        row_base = sid * rows_w
        bufs = ((src0_v, dst0_v, msg0_v, gsem0, ssem0),
                (src1_v, dst1_v, msg1_v, gsem1, ssem1))

        def fire_gathers(hw_ref, src_v, msg_v, gsem):
            for j in range(ch_rows):
                pltpu.async_copy(hw_ref.at[src_v.at[j]],
                                 msg_v.at[pl.ds(j * ROW, ROW)], gsem)

        @pl.loop(0, n_chunks // 2)
        def _(g):
            for b in (0, 1):
                src_v, dst_v, msg_v, gsem, _ = bufs[b]
                rb = row_base + (2 * g + b) * ch_rows
                pltpu.sync_copy(src_hbm.at[pl.ds(rb, ch_rows)], src_v)
                pltpu.sync_copy(dst_hbm.at[pl.ds(rb, ch_rows)], dst_v)

                @pl.when(cid == 0)
                def _():
                    fire_gathers(hw0_hbm, src_v, msg_v, gsem)

                @pl.when(cid == 1)
                def _():
                    fire_gathers(hw1_hbm, src_v, msg_v, gsem)

            scopies = []
            for b in (0, 1):
                src_v, dst_v, msg_v, gsem, ssem = bufs[b]
                for j in range(ch_rows):
                    pltpu.make_async_copy(
                        hw0_hbm.at[src_v.at[j]],
                        msg_v.at[pl.ds(j * ROW, ROW)], gsem).wait()
                for j in range(ch_rows):
                    scopies.append(pltpu.async_copy(
                        msg_v.at[pl.ds(j * ROW, ROW)],
                        acc_sh.at[dst_v.at[j]], ssem, add=True))
            for c in scopies:
                c.wait()

        plsc.subcore_barrier()
        out_off = cid * n_pad + base_n
        pltpu.sync_copy(acc_sh.at[pl.ds(base_n, n_slice)],
                        out_hbm.at[pl.ds(out_off, n_slice)])

    return layer_kernel


def _tc_prep(deg4, x3, embp, w0, n_pad):
    nb = n_pad // BLK

    def body(deg_ref, x_ref, emb_ref, w0_ref, hw_ref, dis_ref):
        deg = deg_ref[0, 0] + deg_ref[1, 0] + 1.0
        dis = lax.rsqrt(deg)
        dis_ref[0] = dis
        xv = x_ref[0]
        cls = lax.broadcasted_iota(I32, (BLK, 128), 1)
        oh = (xv == cls).astype(F32)
        h0 = jnp.dot(oh, emb_ref[...], preferred_element_type=F32)
        th = dis * h0
        hw1 = jnp.dot(th, w0_ref[...], preferred_element_type=F32)
        hw_ref[0] = hw1[:, :16]
        hw_ref[1] = hw1[:, 16:]

    return pl.pallas_call(
        body,
        grid=(nb,),
        in_specs=[
            pl.BlockSpec((2, 1, BLK, 1), lambda i: (0, i, 0, 0)),
            pl.BlockSpec((1, BLK, 1), lambda i: (i, 0, 0)),
            pl.BlockSpec((128, 32), lambda i: (0, 0)),
            pl.BlockSpec((32, 32), lambda i: (0, 0)),
        ],
        out_specs=[
            pl.BlockSpec((2, BLK, 16), lambda i: (0, i, 0)),
            pl.BlockSpec((1, BLK, 1), lambda i: (i, 0, 0)),
        ],
        out_shape=[
            jax.ShapeDtypeStruct((2, n_pad, 16), F32),
            jax.ShapeDtypeStruct((nb, BLK, 1), F32),
        ],
    )(deg4, x3, embp, w0)


def _tc_layer(acc, hwp, dis3, bvec, wnext, n_pad):
    nb = n_pad // BLK

    def body(acc_ref, hw_ref, dis_ref, b_ref, w_ref, out_ref):
        s = jnp.concatenate(
            [acc_ref[0] + hw_ref[0], acc_ref[1] + hw_ref[1]], axis=1)
        d = dis_ref[0]
        h = jnp.maximum(d * s + b_ref[...], 0.0)
        o = jnp.dot(d * h, w_ref[...], preferred_element_type=F32)
        out_ref[0] = o[:, :16]
        out_ref[1] = o[:, 16:]

    return pl.pallas_call(
        body,
        grid=(nb,),
        in_specs=[
            pl.BlockSpec((2, BLK, 16), lambda i: (0, i, 0)),
            pl.BlockSpec((2, BLK, 16), lambda i: (0, i, 0)),
            pl.BlockSpec((1, BLK, 1), lambda i: (i, 0, 0)),
            pl.BlockSpec((1, 32), lambda i: (0, 0)),
            pl.BlockSpec((32, 32), lambda i: (0, 0)),
        ],
        out_specs=pl.BlockSpec((2, BLK, 16), lambda i: (0, i, 0)),
        out_shape=jax.ShapeDtypeStruct((2, n_pad, 16), F32),
    )(acc, hwp, dis3, bvec, wnext)


def _lstm_step1(b_ih_ref, b_hh_ref):
    gb = b_ih_ref[...] + b_hh_ref[...]
    ii = jax.nn.sigmoid(gb[:, 0:32])
    ff = jax.nn.sigmoid(gb[:, 32:64])
    gg = jnp.tanh(gb[:, 64:96])
    oo = jax.nn.sigmoid(gb[:, 96:128])
    c1 = ii * gg
    h1 = oo * jnp.tanh(c1)
    del ff
    return c1, h1


def _block_h3(acc_ref, hw_ref, dis_ref, b2_ref):
    s = jnp.concatenate(
        [acc_ref[0] + hw_ref[0], acc_ref[1] + hw_ref[1]], axis=1)
    d = dis_ref[0]
    return jnp.maximum(d * s + b2_ref[...], 0.0)


def _block_onehot(batch_ref):
    bv = batch_ref[0]
    seg = lax.broadcasted_iota(I32, (BLK, NSEG), 1)
    return bv == seg


def _seg_sweep_a(h3, oh, q):
    hq = lax.dot_general(h3, q, (((1,), (1,)), ((), ())),
                         preferred_element_type=F32)
    masked = jnp.where(oh, hq, -jnp.inf)
    return jnp.max(masked, axis=0, keepdims=True)


def _seg_sweep_b(h3, oh, q, emax):
    ohf = oh.astype(F32)
    hq = lax.dot_general(h3, q, (((1,), (1,)), ((), ())),
                         preferred_element_type=F32)
    e = jnp.sum(ohf * hq, axis=1, keepdims=True)
    emaxf = jnp.where(jnp.isfinite(emax), emax, 0.0)
    emv = lax.dot_general(ohf, emaxf, (((1,), (1,)), ((), ())),
                          preferred_element_type=F32)
    ee = jnp.exp(e - emv) * jnp.sum(ohf, axis=1, keepdims=True)
    esum = jnp.sum(ohf * ee, axis=0, keepdims=True)
    rnum = lax.dot_general(ohf, ee * h3, (((0,), (0,)), ((), ())),
                           preferred_element_type=F32)
    return esum, rnum


_S2S_SPECS = [
    pl.BlockSpec((2, BLK, 16), lambda i: (0, i, 0)),
    pl.BlockSpec((2, BLK, 16), lambda i: (0, i, 0)),
    pl.BlockSpec((1, BLK, 1), lambda i: (i, 0, 0)),
    pl.BlockSpec((1, BLK, 1), lambda i: (i, 0, 0)),
    pl.BlockSpec((1, 32), lambda i: (0, 0)),
]


def _tc_s2s_a(acc, hwp, dis3, batch3, b2, b_ih, b_hh, n_pad):
    nb = n_pad // BLK

    def body(acc_ref, hw_ref, dis_ref, bat_ref, b2_ref, bih_ref, bhh_ref,
             emax_ref):
        i = pl.program_id(0)

        @pl.when(i == 0)
        def _():
            emax_ref[...] = jnp.full((1, NSEG), -jnp.inf, F32)

        _, h1 = _lstm_step1(bih_ref, bhh_ref)
        q1 = jnp.broadcast_to(h1, (NSEG, 32))
        h3 = _block_h3(acc_ref, hw_ref, dis_ref, b2_ref)
        oh = _block_onehot(bat_ref)
        emax_ref[...] = jnp.maximum(emax_ref[...], _seg_sweep_a(h3, oh, q1))

    return pl.pallas_call(
        body,
        grid=(nb,),
        in_specs=_S2S_SPECS + [
            pl.BlockSpec((1, 128), lambda i: (0, 0)),
            pl.BlockSpec((1, 128), lambda i: (0, 0)),
        ],
        out_specs=pl.BlockSpec((1, NSEG), lambda i: (0, 0)),
        out_shape=jax.ShapeDtypeStruct((1, NSEG), F32),
    )(acc, hwp, dis3, batch3, b2, b_ih, b_hh)


def _tc_s2s_b(acc, hwp, dis3, batch3, b2, b_ih, b_hh, emax1, n_pad):
    nb = n_pad // BLK

    def body(acc_ref, hw_ref, dis_ref, bat_ref, b2_ref, bih_ref, bhh_ref,
             emax_ref, esum_ref, rnum_ref):
        i = pl.program_id(0)

        @pl.when(i == 0)
        def _():
            esum_ref[...] = jnp.zeros((1, NSEG), F32)
            rnum_ref[...] = jnp.zeros((NSEG, 32), F32)

        _, h1 = _lstm_step1(bih_ref, bhh_ref)
        q1 = jnp.broadcast_to(h1, (NSEG, 32))
        h3 = _block_h3(acc_ref, hw_ref, dis_ref, b2_ref)
        oh = _block_onehot(bat_ref)
        esum, rnum = _seg_sweep_b(h3, oh, q1, emax_ref[...])
        esum_ref[...] += esum
        rnum_ref[...] += rnum

    return pl.pallas_call(
        body,
        grid=(nb,),
        in_specs=_S2S_SPECS + [
            pl.BlockSpec((1, 128), lambda i: (0, 0)),
            pl.BlockSpec((1, 128), lambda i: (0, 0)),
            pl.BlockSpec((1, NSEG), lambda i: (0, 0)),
        ],
        out_specs=[
            pl.BlockSpec((1, NSEG), lambda i: (0, 0)),
            pl.BlockSpec((NSEG, 32), lambda i: (0, 0)),
        ],
        out_shape=[
            jax.ShapeDtypeStruct((1, NSEG), F32),
            jax.ShapeDtypeStruct((NSEG, 32), F32),
        ],
    )(acc, hwp, dis3, batch3, b2, b_ih, b_hh, emax1)


def _lstm_step2(bih_ref, bhh_ref, wih_ref, whh_ref, esum1, rnum1):
    c1, h1 = _lstm_step1(bih_ref, bhh_ref)
    q1 = jnp.broadcast_to(h1, (NSEG, 32))
    r1 = rnum1 / (jnp.transpose(esum1) + 1e-16)
    qs1 = jnp.concatenate([q1, r1], axis=1)
    gates = (jnp.dot(qs1, wih_ref[...], preferred_element_type=F32)
             + bih_ref[...]
             + jnp.dot(jnp.broadcast_to(h1, (NSEG, 32)), whh_ref[...],
                       preferred_element_type=F32)
             + bhh_ref[...])
    i2 = jax.nn.sigmoid(gates[:, 0:32])
    f2 = jax.nn.sigmoid(gates[:, 32:64])
    g2 = jnp.tanh(gates[:, 64:96])
    o2 = jax.nn.sigmoid(gates[:, 96:128])
    c2 = f2 * c1 + i2 * g2
    h2 = o2 * jnp.tanh(c2)
    return h2


def _tc_s2s_c(acc, hwp, dis3, batch3, b2, b_ih, b_hh, wihT, whhT,
              esum1, rnum1, n_pad):
    nb = n_pad // BLK

    def body(acc_ref, hw_ref, dis_ref, bat_ref, b2_ref, bih_ref, bhh_ref,
             wih_ref, whh_ref, esum_ref, rnum_ref, emax_ref, q2_ref):
        i = pl.program_id(0)

        @pl.when(i == 0)
        def _():
            emax_ref[...] = jnp.full((1, NSEG), -jnp.inf, F32)

        q2 = _lstm_step2(bih_ref, bhh_ref, wih_ref, whh_ref,
                         esum_ref[...], rnum_ref[...])
        q2_ref[...] = q2
        h3 = _block_h3(acc_ref, hw_ref, dis_ref, b2_ref)
        oh = _block_onehot(bat_ref)
        emax_ref[...] = jnp.maximum(emax_ref[...], _seg_sweep_a(h3, oh, q2))

    return pl.pallas_call(
        body,
        grid=(nb,),
        in_specs=_S2S_SPECS + [
            pl.BlockSpec((1, 128), lambda i: (0, 0)),
            pl.BlockSpec((1, 128), lambda i: (0, 0)),
            pl.BlockSpec((64, 128), lambda i: (0, 0)),
            pl.BlockSpec((32, 128), lambda i: (0, 0)),
            pl.BlockSpec((1, NSEG), lambda i: (0, 0)),
            pl.BlockSpec((NSEG, 32), lambda i: (0, 0)),
        ],
        out_specs=[
            pl.BlockSpec((1, NSEG), lambda i: (0, 0)),
            pl.BlockSpec((NSEG, 32), lambda i: (0, 0)),
        ],
        out_shape=[
            jax.ShapeDtypeStruct((1, NSEG), F32),
            jax.ShapeDtypeStruct((NSEG, 32), F32),
        ],
    )(acc, hwp, dis3, batch3, b2, b_ih, b_hh, wihT, whhT, esum1, rnum1)


def _tc_s2s_d(acc, hwp, dis3, batch3, b2, emax2, q2, lin0WT, lin0b,
              lin3WT, lin3b, n_pad):
    nb = n_pad // BLK

    def body(acc_ref, hw_ref, dis_ref, bat_ref, b2_ref, emax_ref, q2_ref,
             l0w_ref, l0b_ref, l3w_ref, l3b_ref, out_ref,
             esum_s, rnum_s):
        i = pl.program_id(0)

        @pl.when(i == 0)
        def _():
            esum_s[...] = jnp.zeros((1, NSEG), F32)
            rnum_s[...] = jnp.zeros((NSEG, 32), F32)

        q2 = q2_ref[...]
        h3 = _block_h3(acc_ref, hw_ref, dis_ref, b2_ref)
        oh = _block_onehot(bat_ref)
        esum, rnum = _seg_sweep_b(h3, oh, q2, emax_ref[...])
        esum_s[...] += esum
        rnum_s[...] += rnum

        @pl.when(i == nb - 1)
        def _():
            r2 = rnum_s[...] / (jnp.transpose(esum_s[...]) + 1e-16)
            qs2 = jnp.concatenate([q2, r2], axis=1)
            z = jnp.maximum(
                jnp.dot(qs2, l0w_ref[...], preferred_element_type=F32)
                + l0b_ref[...], 0.0)
            out_ref[...] = (jnp.dot(z, l3w_ref[...],
                                    preferred_element_type=F32)
                            + l3b_ref[...])

    return pl.pallas_call(
        body,
        grid=(nb,),
        in_specs=_S2S_SPECS + [
            pl.BlockSpec((1, NSEG), lambda i: (0, 0)),
            pl.BlockSpec((NSEG, 32), lambda i: (0, 0)),
            pl.BlockSpec((64, 32), lambda i: (0, 0)),
            pl.BlockSpec((1, 32), lambda i: (0, 0)),
            pl.BlockSpec((32, 1), lambda i: (0, 0)),
            pl.BlockSpec((1, 1), lambda i: (0, 0)),
        ],
        out_specs=pl.BlockSpec((NSEG, 1), lambda i: (0, 0)),
        out_shape=jax.ShapeDtypeStruct((NSEG, 1), F32),
        scratch_shapes=[
            pltpu.VMEM((1, NSEG), F32),
            pltpu.VMEM((NSEG, 32), F32),
        ],
    )(acc, hwp, dis3, batch3, b2, emax2, q2, lin0WT, lin0b, lin3WT, lin3b)


def kernel(x, edge_index, batch, emb, W0, b0, W1, b1, W2, b2,
           W_ih, W_hh, b_ih, b_hh, lin0_W, lin0_b, lin3_W, lin3_b):
    n = x.shape[0]
    e = edge_index.shape[1]

    n_pad = _ceil_to(n, NSUB * ROW)
    if n_pad == n:
        n_pad += NSUB * ROW
    pad_rows = n_pad - n
    e_pad = _ceil_to(e, NCORES * NSUB * 16 * ROW)
    nb = n_pad // BLK

    src = edge_index[0].astype(I32)
    dst = edge_index[1].astype(I32)
    pe = e_pad - e
    pad_ar = jnp.arange(pe, dtype=I32)
    src_p = jnp.concatenate([src, (pad_ar * 97) % n])
    dst_p = jnp.concatenate([dst, n + pad_ar % pad_rows])
    src2d = src_p.reshape(e_pad // ROW, ROW)
    dst2d = dst_p.reshape(e_pad // ROW, ROW)

    x_p = jnp.concatenate([x.astype(I32), jnp.zeros((n_pad - n,), I32)])
    x3 = x_p.reshape(nb, BLK, 1)
    batch_p = jnp.concatenate(
        [batch.astype(I32), jnp.full((n_pad - n,), NSEG, I32)])
    batch3 = batch_p.reshape(nb, BLK, 1)

    embp = jnp.zeros((128, 32), F32).at[:emb.shape[0]].set(emb)
    b0r = b0.reshape(1, 32)
    b1r = b1.reshape(1, 32)
    b2r = b2.reshape(1, 32)
    bihr = b_ih.reshape(1, 128)
    bhhr = b_hh.reshape(1, 128)
    wihT = W_ih.T
    whhT = W_hh.T
    lin0WT = lin0_W.T
    lin0br = lin0_b.reshape(1, 32)
    lin3WT = lin3_W.T
    lin3br = lin3_b.reshape(1, 1)

    deg_flat = _make_deg_kernel(e_pad, n_pad)(dst2d)
    deg4 = deg_flat.reshape(2, nb, BLK, 1)

    hw1, dis3 = _tc_prep(deg4, x3, embp, W0, n_pad)

    layer = _make_layer_kernel(e_pad, n_pad)

    acc1 = layer(hw1[0], hw1[1], src2d, dst2d).reshape(2, n_pad, 16)
    hw2 = _tc_layer(acc1, hw1, dis3, b0r, W1, n_pad)
    acc2 = layer(hw2[0], hw2[1], src2d, dst2d).reshape(2, n_pad, 16)
    hw3 = _tc_layer(acc2, hw2, dis3, b1r, W2, n_pad)
    acc3 = layer(hw3[0], hw3[1], src2d, dst2d).reshape(2, n_pad, 16)

    emax1 = _tc_s2s_a(acc3, hw3, dis3, batch3, b2r, bihr, bhhr, n_pad)
    esum1, rnum1 = _tc_s2s_b(acc3, hw3, dis3, batch3, b2r, bihr, bhhr,
                             emax1, n_pad)
    emax2, q2 = _tc_s2s_c(acc3, hw3, dis3, batch3, b2r, bihr, bhhr,
                          wihT, whhT, esum1, rnum1, n_pad)
    out = _tc_s2s_d(acc3, hw3, dis3, batch3, b2r, emax2, q2,
                    lin0WT, lin0br, lin3WT, lin3br, n_pad)
    return out


# speedup vs baseline: 21.9014x; 1.2314x over previous
import functools

import jax
import jax.numpy as jnp
from jax import lax
from jax.experimental import pallas as pl
from jax.experimental.pallas import tpu as pltpu
from jax.experimental.pallas import tpu_sc as plsc

F32 = jnp.float32
I32 = jnp.int32

NCORES = 2
NSUB = 16
ROW = 128
BLK = 1024
NSEG = 256

_SC_PARAMS = pltpu.CompilerParams(use_tc_tiling_on_sc=False)


def _ceil_to(a, m):
    return (a + m - 1) // m * m


def _make_deg_kernel(e_pad, n_pad):
    rows_w = e_pad // ROW // (NCORES * NSUB)
    n_slice = n_pad // NSUB
    mesh = plsc.VectorSubcoreMesh(core_axis_name="c", subcore_axis_name="s")

    @functools.partial(
        pl.kernel,
        mesh=mesh,
        compiler_params=_SC_PARAMS,
        out_type=jax.ShapeDtypeStruct((NCORES * n_pad,), F32),
        scratch_types=[
            pltpu.VMEM((rows_w, ROW), I32),
            pltpu.VMEM((n_slice,), F32),
            pltpu.VMEM((ROW,), F32),
            pltpu.VMEM_SHARED((n_pad,), F32),
        ],
    )
    def deg_kernel(dst_hbm, out_hbm, idx_v, zero_v, ones_v, acc_sh):
        cid = lax.axis_index("c")
        sid = lax.axis_index("s")

        @pl.loop(0, n_slice // 16)
        def _(i):
            zero_v[pl.ds(i * 16, 16)] = jnp.zeros((16,), F32)

        @pl.loop(0, ROW // 16)
        def _(i):
            ones_v[pl.ds(i * 16, 16)] = jnp.ones((16,), F32)

        pltpu.sync_copy(zero_v, acc_sh.at[pl.ds(sid * n_slice, n_slice)])
        plsc.subcore_barrier()

        wid = cid * NSUB + sid
        pltpu.sync_copy(dst_hbm.at[pl.ds(wid * rows_w, rows_w)], idx_v)

        @pl.loop(0, rows_w)
        def _(j):
            pltpu.sync_copy(ones_v, acc_sh.at[idx_v.at[j]], add=True)

        plsc.subcore_barrier()
        out_off = cid * n_pad + sid * n_slice
        pltpu.sync_copy(acc_sh.at[pl.ds(sid * n_slice, n_slice)],
                        out_hbm.at[pl.ds(out_off, n_slice)])

    return deg_kernel


def _make_layer_kernel(e_pad, n_pad):
    rows_w = e_pad // ROW // NSUB
    ch_rows = 4
    n_chunks = rows_w // ch_rows
    ch_e = ch_rows * ROW
    n_slice = n_pad // NSUB
    mesh = plsc.VectorSubcoreMesh(core_axis_name="c", subcore_axis_name="s")

    idx_t = pltpu.VMEM((ch_rows, ROW), I32)
    msg_t = pltpu.VMEM((ch_e, 16), F32)
    sem_t = pltpu.SemaphoreType.DMA

    @functools.partial(
        pl.kernel,
        mesh=mesh,
        compiler_params=_SC_PARAMS,
        out_type=jax.ShapeDtypeStruct((NCORES * n_pad, 16), F32),
        scratch_types=(
            [idx_t] * 6 + [msg_t] * 3 + [sem_t] * 9
            + [pltpu.VMEM_SHARED((n_pad, 16), F32)]
        ),
    )
    def layer_kernel(hw0_hbm, hw1_hbm, src_hbm, dst_hbm, out_hbm,
                     s0, s1, s2, d0, d1, d2, m0, m1, m2,
                     i0, i1, i2, g0, g1, g2, t0, t1, t2, acc_sh):
        cid = lax.axis_index("c")
        sid = lax.axis_index("s")
        srcs, dsts, msgs = (s0, s1, s2), (d0, d1, d2), (m0, m1, m2)
        isems, gsems, ssems = (i0, i1, i2), (g0, g1, g2), (t0, t1, t2)

        @pl.loop(0, ch_e)
        def _(i):
            m0[i] = jnp.zeros((16,), F32)

        base_n = sid * n_slice

        @pl.loop(0, n_slice // ch_e)
        def _(i):
            pltpu.sync_copy(m0, acc_sh.at[pl.ds(base_n + i * ch_e, ch_e)])

        rem = n_slice % ch_e
        if rem:
            pltpu.sync_copy(m0.at[pl.ds(0, rem)],
                            acc_sh.at[pl.ds(base_n + (n_slice // ch_e) * ch_e,
                                            rem)])
        plsc.subcore_barrier()

        row_base = sid * rows_w

        def fire_idx(c, b):
            rb = row_base + c * ch_rows
            pltpu.async_copy(src_hbm.at[pl.ds(rb, ch_rows)], srcs[b], isems[b])
            pltpu.async_copy(dst_hbm.at[pl.ds(rb, ch_rows)], dsts[b], isems[b])

        def drain_idx(b):
            for _ in range(2):
                pltpu.make_async_copy(src_hbm.at[pl.ds(0, ch_rows)],
                                      srcs[b], isems[b]).wait()

        def fire_gathers(b):
            @pl.when(cid == 0)
            def _():
                for j in range(ch_rows):
                    pltpu.async_copy(hw0_hbm.at[srcs[b].at[j]],
                                     msgs[b].at[pl.ds(j * ROW, ROW)],
                                     gsems[b])

            @pl.when(cid == 1)
            def _():
                for j in range(ch_rows):
                    pltpu.async_copy(hw1_hbm.at[srcs[b].at[j]],
                                     msgs[b].at[pl.ds(j * ROW, ROW)],
                                     gsems[b])

        def drain_gathers(b):
            for j in range(ch_rows):
                pltpu.make_async_copy(
                    hw0_hbm.at[srcs[b].at[j]],
                    msgs[b].at[pl.ds(j * ROW, ROW)], gsems[b]).wait()

        def fire_scatters(b):
            for j in range(ch_rows):
                pltpu.async_copy(msgs[b].at[pl.ds(j * ROW, ROW)],
                                 acc_sh.at[dsts[b].at[j]], ssems[b], add=True)

        def drain_scatters(b):
            for j in range(ch_rows):
                pltpu.make_async_copy(
                    msgs[b].at[pl.ds(j * ROW, ROW)],
                    acc_sh.at[dsts[b].at[j]], ssems[b]).wait()

        def ring_step(c, b, pb, nb, stage):
            drain_idx(b)
            fire_gathers(b)
            if stage >= 1:
                drain_gathers(pb)
                fire_scatters(pb)
            if stage >= 2:
                drain_scatters(nb)
            fire_idx(c + 1, nb)

        fire_idx(0, 0)
        ring_step(0, 0, 2, 1, 0)
        ring_step(1, 1, 0, 2, 1)
        ring_step(2, 2, 1, 0, 2)

        @pl.loop(1, n_chunks // 3)
        def _(g):
            c = g * 3
            ring_step(c, 0, 2, 1, 2)
            ring_step(c + 1, 1, 0, 2, 2)
            ring_step(c + 2, 2, 1, 0, 2)

        drain_gathers(2)
        fire_scatters(2)
        drain_scatters(1)
        drain_scatters(2)
        drain_idx(0)

        plsc.subcore_barrier()
        out_off = cid * n_pad + base_n
        pltpu.sync_copy(acc_sh.at[pl.ds(base_n, n_slice)],
                        out_hbm.at[pl.ds(out_off, n_slice)])

    return layer_kernel


def _tc_prep(deg4, x3, embp, w0, n_pad):
    nb = n_pad // BLK

    def body(deg_ref, x_ref, emb_ref, w0_ref, hw_ref, dis_ref):
        deg = deg_ref[0, 0] + deg_ref[1, 0] + 1.0
        dis = lax.rsqrt(deg)
        dis_ref[0] = dis
        xv = x_ref[0]
        cls = lax.broadcasted_iota(I32, (BLK, 128), 1)
        oh = (xv == cls).astype(F32)
        h0 = jnp.dot(oh, emb_ref[...], preferred_element_type=F32)
        th = dis * h0
        hw1 = jnp.dot(th, w0_ref[...], preferred_element_type=F32)
        hw_ref[0] = hw1[:, :16]
        hw_ref[1] = hw1[:, 16:]

    return pl.pallas_call(
        body,
        grid=(nb,),
        in_specs=[
            pl.BlockSpec((2, 1, BLK, 1), lambda i: (0, i, 0, 0)),
            pl.BlockSpec((1, BLK, 1), lambda i: (i, 0, 0)),
            pl.BlockSpec((128, 32), lambda i: (0, 0)),
            pl.BlockSpec((32, 32), lambda i: (0, 0)),
        ],
        out_specs=[
            pl.BlockSpec((2, BLK, 16), lambda i: (0, i, 0)),
            pl.BlockSpec((1, BLK, 1), lambda i: (i, 0, 0)),
        ],
        out_shape=[
            jax.ShapeDtypeStruct((2, n_pad, 16), F32),
            jax.ShapeDtypeStruct((nb, BLK, 1), F32),
        ],
    )(deg4, x3, embp, w0)


def _tc_layer(acc, hwp, dis3, bvec, wnext, n_pad):
    nb = n_pad // BLK

    def body(acc_ref, hw_ref, dis_ref, b_ref, w_ref, out_ref):
        s = jnp.concatenate(
            [acc_ref[0] + hw_ref[0], acc_ref[1] + hw_ref[1]], axis=1)
        d = dis_ref[0]
        h = jnp.maximum(d * s + b_ref[...], 0.0)
        o = jnp.dot(d * h, w_ref[...], preferred_element_type=F32)
        out_ref[0] = o[:, :16]
        out_ref[1] = o[:, 16:]

    return pl.pallas_call(
        body,
        grid=(nb,),
        in_specs=[
            pl.BlockSpec((2, BLK, 16), lambda i: (0, i, 0)),
            pl.BlockSpec((2, BLK, 16), lambda i: (0, i, 0)),
            pl.BlockSpec((1, BLK, 1), lambda i: (i, 0, 0)),
            pl.BlockSpec((1, 32), lambda i: (0, 0)),
            pl.BlockSpec((32, 32), lambda i: (0, 0)),
        ],
        out_specs=pl.BlockSpec((2, BLK, 16), lambda i: (0, i, 0)),
        out_shape=jax.ShapeDtypeStruct((2, n_pad, 16), F32),
    )(acc, hwp, dis3, bvec, wnext)


def _lstm_step1(b_ih_ref, b_hh_ref):
    gb = b_ih_ref[...] + b_hh_ref[...]
    ii = jax.nn.sigmoid(gb[:, 0:32])
    ff = jax.nn.sigmoid(gb[:, 32:64])
    gg = jnp.tanh(gb[:, 64:96])
    oo = jax.nn.sigmoid(gb[:, 96:128])
    c1 = ii * gg
    h1 = oo * jnp.tanh(c1)
    del ff
    return c1, h1


def _block_h3(acc_ref, hw_ref, dis_ref, b2_ref):
    s = jnp.concatenate(
        [acc_ref[0] + hw_ref[0], acc_ref[1] + hw_ref[1]], axis=1)
    d = dis_ref[0]
    return jnp.maximum(d * s + b2_ref[...], 0.0)


def _block_onehot(batch_ref):
    bv = batch_ref[0]
    seg = lax.broadcasted_iota(I32, (BLK, NSEG), 1)
    return bv == seg


def _seg_sweep_a(h3, oh, q):
    hq = lax.dot_general(h3, q, (((1,), (1,)), ((), ())),
                         preferred_element_type=F32)
    masked = jnp.where(oh, hq, -jnp.inf)
    return jnp.max(masked, axis=0, keepdims=True)


def _seg_sweep_b(h3, oh, q, emax):
    ohf = oh.astype(F32)
    hq = lax.dot_general(h3, q, (((1,), (1,)), ((), ())),
                         preferred_element_type=F32)
    e = jnp.sum(ohf * hq, axis=1, keepdims=True)
    emaxf = jnp.where(jnp.isfinite(emax), emax, 0.0)
    emv = lax.dot_general(ohf, emaxf, (((1,), (1,)), ((), ())),
                          preferred_element_type=F32)
    ee = jnp.exp(e - emv) * jnp.sum(ohf, axis=1, keepdims=True)
    esum = jnp.sum(ohf * ee, axis=0, keepdims=True)
    rnum = lax.dot_general(ohf, ee * h3, (((0,), (0,)), ((), ())),
                           preferred_element_type=F32)
    return esum, rnum


_S2S_SPECS = [
    pl.BlockSpec((2, BLK, 16), lambda i: (0, i, 0)),
    pl.BlockSpec((2, BLK, 16), lambda i: (0, i, 0)),
    pl.BlockSpec((1, BLK, 1), lambda i: (i, 0, 0)),
    pl.BlockSpec((1, BLK, 1), lambda i: (i, 0, 0)),
    pl.BlockSpec((1, 32), lambda i: (0, 0)),
]


def _tc_s2s_a(acc, hwp, dis3, batch3, b2, b_ih, b_hh, n_pad):
    nb = n_pad // BLK

    def body(acc_ref, hw_ref, dis_ref, bat_ref, b2_ref, bih_ref, bhh_ref,
             emax_ref):
        i = pl.program_id(0)

        @pl.when(i == 0)
        def _():
            emax_ref[...] = jnp.full((1, NSEG), -jnp.inf, F32)

        _, h1 = _lstm_step1(bih_ref, bhh_ref)
        q1 = jnp.broadcast_to(h1, (NSEG, 32))
        h3 = _block_h3(acc_ref, hw_ref, dis_ref, b2_ref)
        oh = _block_onehot(bat_ref)
        emax_ref[...] = jnp.maximum(emax_ref[...], _seg_sweep_a(h3, oh, q1))

    return pl.pallas_call(
        body,
        grid=(nb,),
        in_specs=_S2S_SPECS + [
            pl.BlockSpec((1, 128), lambda i: (0, 0)),
            pl.BlockSpec((1, 128), lambda i: (0, 0)),
        ],
        out_specs=pl.BlockSpec((1, NSEG), lambda i: (0, 0)),
        out_shape=jax.ShapeDtypeStruct((1, NSEG), F32),
    )(acc, hwp, dis3, batch3, b2, b_ih, b_hh)


def _tc_s2s_b(acc, hwp, dis3, batch3, b2, b_ih, b_hh, emax1, n_pad):
    nb = n_pad // BLK

    def body(acc_ref, hw_ref, dis_ref, bat_ref, b2_ref, bih_ref, bhh_ref,
             emax_ref, esum_ref, rnum_ref):
        i = pl.program_id(0)

        @pl.when(i == 0)
        def _():
            esum_ref[...] = jnp.zeros((1, NSEG), F32)
            rnum_ref[...] = jnp.zeros((NSEG, 32), F32)

        _, h1 = _lstm_step1(bih_ref, bhh_ref)
        q1 = jnp.broadcast_to(h1, (NSEG, 32))
        h3 = _block_h3(acc_ref, hw_ref, dis_ref, b2_ref)
        oh = _block_onehot(bat_ref)
        esum, rnum = _seg_sweep_b(h3, oh, q1, emax_ref[...])
        esum_ref[...] += esum
        rnum_ref[...] += rnum

    return pl.pallas_call(
        body,
        grid=(nb,),
        in_specs=_S2S_SPECS + [
            pl.BlockSpec((1, 128), lambda i: (0, 0)),
            pl.BlockSpec((1, 128), lambda i: (0, 0)),
            pl.BlockSpec((1, NSEG), lambda i: (0, 0)),
        ],
        out_specs=[
            pl.BlockSpec((1, NSEG), lambda i: (0, 0)),
            pl.BlockSpec((NSEG, 32), lambda i: (0, 0)),
        ],
        out_shape=[
            jax.ShapeDtypeStruct((1, NSEG), F32),
            jax.ShapeDtypeStruct((NSEG, 32), F32),
        ],
    )(acc, hwp, dis3, batch3, b2, b_ih, b_hh, emax1)


def _lstm_step2(bih_ref, bhh_ref, wih_ref, whh_ref, esum1, rnum1):
    c1, h1 = _lstm_step1(bih_ref, bhh_ref)
    q1 = jnp.broadcast_to(h1, (NSEG, 32))
    r1 = rnum1 / (jnp.transpose(esum1) + 1e-16)
    qs1 = jnp.concatenate([q1, r1], axis=1)
    gates = (jnp.dot(qs1, wih_ref[...], preferred_element_type=F32)
             + bih_ref[...]
             + jnp.dot(jnp.broadcast_to(h1, (NSEG, 32)), whh_ref[...],
                       preferred_element_type=F32)
             + bhh_ref[...])
    i2 = jax.nn.sigmoid(gates[:, 0:32])
    f2 = jax.nn.sigmoid(gates[:, 32:64])
    g2 = jnp.tanh(gates[:, 64:96])
    o2 = jax.nn.sigmoid(gates[:, 96:128])
    c2 = f2 * c1 + i2 * g2
    h2 = o2 * jnp.tanh(c2)
    return h2


def _tc_s2s_c(acc, hwp, dis3, batch3, b2, b_ih, b_hh, wihT, whhT,
              esum1, rnum1, n_pad):
    nb = n_pad // BLK

    def body(acc_ref, hw_ref, dis_ref, bat_ref, b2_ref, bih_ref, bhh_ref,
             wih_ref, whh_ref, esum_ref, rnum_ref, emax_ref, q2_ref, q2_s):
        i = pl.program_id(0)

        @pl.when(i == 0)
        def _():
            emax_ref[...] = jnp.full((1, NSEG), -jnp.inf, F32)
            q2_s[...] = _lstm_step2(bih_ref, bhh_ref, wih_ref, whh_ref,
                                    esum_ref[...], rnum_ref[...])
            q2_ref[...] = q2_s[...]

        q2 = q2_s[...]
        h3 = _block_h3(acc_ref, hw_ref, dis_ref, b2_ref)
        oh = _block_onehot(bat_ref)
        emax_ref[...] = jnp.maximum(emax_ref[...], _seg_sweep_a(h3, oh, q2))

    return pl.pallas_call(
        body,
        grid=(nb,),
        in_specs=_S2S_SPECS + [
            pl.BlockSpec((1, 128), lambda i: (0, 0)),
            pl.BlockSpec((1, 128), lambda i: (0, 0)),
            pl.BlockSpec((64, 128), lambda i: (0, 0)),
            pl.BlockSpec((32, 128), lambda i: (0, 0)),
            pl.BlockSpec((1, NSEG), lambda i: (0, 0)),
            pl.BlockSpec((NSEG, 32), lambda i: (0, 0)),
        ],
        out_specs=[
            pl.BlockSpec((1, NSEG), lambda i: (0, 0)),
            pl.BlockSpec((NSEG, 32), lambda i: (0, 0)),
        ],
        out_shape=[
            jax.ShapeDtypeStruct((1, NSEG), F32),
            jax.ShapeDtypeStruct((NSEG, 32), F32),
        ],
        scratch_shapes=[pltpu.VMEM((NSEG, 32), F32)],
    )(acc, hwp, dis3, batch3, b2, b_ih, b_hh, wihT, whhT, esum1, rnum1)


def _tc_s2s_d(acc, hwp, dis3, batch3, b2, emax2, q2, lin0WT, lin0b,
              lin3WT, lin3b, n_pad):
    nb = n_pad // BLK

    def body(acc_ref, hw_ref, dis_ref, bat_ref, b2_ref, emax_ref, q2_ref,
             l0w_ref, l0b_ref, l3w_ref, l3b_ref, out_ref,
             esum_s, rnum_s):
        i = pl.program_id(0)

        @pl.when(i == 0)
        def _():
            esum_s[...] = jnp.zeros((1, NSEG), F32)
            rnum_s[...] = jnp.zeros((NSEG, 32), F32)

        q2 = q2_ref[...]
        h3 = _block_h3(acc_ref, hw_ref, dis_ref, b2_ref)
        oh = _block_onehot(bat_ref)
        esum, rnum = _seg_sweep_b(h3, oh, q2, emax_ref[...])
        esum_s[...] += esum
        rnum_s[...] += rnum

        @pl.when(i == nb - 1)
        def _():
            r2 = rnum_s[...] / (jnp.transpose(esum_s[...]) + 1e-16)
            qs2 = jnp.concatenate([q2, r2], axis=1)
            z = jnp.maximum(
                jnp.dot(qs2, l0w_ref[...], preferred_element_type=F32)
                + l0b_ref[...], 0.0)
            out_ref[...] = (jnp.dot(z, l3w_ref[...],
                                    preferred_element_type=F32)
                            + l3b_ref[...])

    return pl.pallas_call(
        body,
        grid=(nb,),
        in_specs=_S2S_SPECS + [
            pl.BlockSpec((1, NSEG), lambda i: (0, 0)),
            pl.BlockSpec((NSEG, 32), lambda i: (0, 0)),
            pl.BlockSpec((64, 32), lambda i: (0, 0)),
            pl.BlockSpec((1, 32), lambda i: (0, 0)),
            pl.BlockSpec((32, 1), lambda i: (0, 0)),
            pl.BlockSpec((1, 1), lambda i: (0, 0)),
        ],
        out_specs=pl.BlockSpec((NSEG, 1), lambda i: (0, 0)),
        out_shape=jax.ShapeDtypeStruct((NSEG, 1), F32),
        scratch_shapes=[
            pltpu.VMEM((1, NSEG), F32),
            pltpu.VMEM((NSEG, 32), F32),
        ],
    )(acc, hwp, dis3, batch3, b2, emax2, q2, lin0WT, lin0b, lin3WT, lin3b)


def kernel(x, edge_index, batch, emb, W0, b0, W1, b1, W2, b2,
           W_ih, W_hh, b_ih, b_hh, lin0_W, lin0_b, lin3_W, lin3_b):
    n = x.shape[0]
    e = edge_index.shape[1]

    n_pad = _ceil_to(n, NSUB * ROW)
    if n_pad == n:
        n_pad += NSUB * ROW
    pad_rows = n_pad - n
    e_pad = _ceil_to(e, NSUB * ROW * 12)
    nb = n_pad // BLK

    src = edge_index[0].astype(I32)
    dst = edge_index[1].astype(I32)
    pe = e_pad + 4 * ROW - e
    pad_ar = jnp.arange(pe, dtype=I32)
    src_p = jnp.concatenate([src, (pad_ar * 97) % n])
    dst_p = jnp.concatenate([dst, n + pad_ar % pad_rows])
    src2d = src_p.reshape(e_pad // ROW + 4, ROW)
    dst2d = dst_p.reshape(e_pad // ROW + 4, ROW)

    x_p = jnp.concatenate([x.astype(I32), jnp.zeros((n_pad - n,), I32)])
    x3 = x_p.reshape(nb, BLK, 1)
    batch_p = jnp.concatenate(
        [batch.astype(I32), jnp.full((n_pad - n,), NSEG, I32)])
    batch3 = batch_p.reshape(nb, BLK, 1)

    embp = jnp.zeros((128, 32), F32).at[:emb.shape[0]].set(emb)
    b0r = b0.reshape(1, 32)
    b1r = b1.reshape(1, 32)
    b2r = b2.reshape(1, 32)
    bihr = b_ih.reshape(1, 128)
    bhhr = b_hh.reshape(1, 128)
    wihT = W_ih.T
    whhT = W_hh.T
    lin0WT = lin0_W.T
    lin0br = lin0_b.reshape(1, 32)
    lin3WT = lin3_W.T
    lin3br = lin3_b.reshape(1, 1)

    deg_flat = _make_deg_kernel(e_pad, n_pad)(dst2d)
    deg4 = deg_flat.reshape(2, nb, BLK, 1)

    hw1, dis3 = _tc_prep(deg4, x3, embp, W0, n_pad)

    layer = _make_layer_kernel(e_pad, n_pad)

    acc1 = layer(hw1[0], hw1[1], src2d, dst2d).reshape(2, n_pad, 16)
    hw2 = _tc_layer(acc1, hw1, dis3, b0r, W1, n_pad)
    acc2 = layer(hw2[0], hw2[1], src2d, dst2d).reshape(2, n_pad, 16)
    hw3 = _tc_layer(acc2, hw2, dis3, b1r, W2, n_pad)
    acc3 = layer(hw3[0], hw3[1], src2d, dst2d).reshape(2, n_pad, 16)

    emax1 = _tc_s2s_a(acc3, hw3, dis3, batch3, b2r, bihr, bhhr, n_pad)
    esum1, rnum1 = _tc_s2s_b(acc3, hw3, dis3, batch3, b2r, bihr, bhhr,
                             emax1, n_pad)
    emax2, q2 = _tc_s2s_c(acc3, hw3, dis3, batch3, b2r, bihr, bhhr,
                          wihT, whhT, esum1, rnum1, n_pad)
    out = _tc_s2s_d(acc3, hw3, dis3, batch3, b2r, emax2, q2,
                    lin0WT, lin0br, lin3WT, lin3br, n_pad)
    return out


# speedup vs baseline: 24.3453x; 1.1116x over previous
import functools

import jax
import jax.numpy as jnp
from jax import lax
from jax.experimental import pallas as pl
from jax.experimental.pallas import tpu as pltpu
from jax.experimental.pallas import tpu_sc as plsc

F32 = jnp.float32
I32 = jnp.int32

NCORES = 2
NSUB = 16
ROW = 128
BLK = 1024
NSEG = 256

_SC_PARAMS = pltpu.CompilerParams(use_tc_tiling_on_sc=False)


def _ceil_to(a, m):
    return (a + m - 1) // m * m


def _make_deg_kernel(e_pad, n_pad):
    rows_w = e_pad // ROW // (NCORES * NSUB)
    n_slice = n_pad // NSUB
    mesh = plsc.VectorSubcoreMesh(core_axis_name="c", subcore_axis_name="s")

    @functools.partial(
        pl.kernel,
        mesh=mesh,
        compiler_params=_SC_PARAMS,
        out_type=jax.ShapeDtypeStruct((NCORES * n_pad,), F32),
        scratch_types=[
            pltpu.VMEM((rows_w, ROW), I32),
            pltpu.VMEM((n_slice,), F32),
            pltpu.VMEM((ROW,), F32),
            pltpu.VMEM_SHARED((n_pad,), F32),
        ],
    )
    def deg_kernel(dst_hbm, out_hbm, idx_v, zero_v, ones_v, acc_sh):
        cid = lax.axis_index("c")
        sid = lax.axis_index("s")

        @pl.loop(0, n_slice // 16)
        def _(i):
            zero_v[pl.ds(i * 16, 16)] = jnp.zeros((16,), F32)

        @pl.loop(0, ROW // 16)
        def _(i):
            ones_v[pl.ds(i * 16, 16)] = jnp.ones((16,), F32)

        pltpu.sync_copy(zero_v, acc_sh.at[pl.ds(sid * n_slice, n_slice)])
        plsc.subcore_barrier()

        wid = cid * NSUB + sid
        pltpu.sync_copy(dst_hbm.at[pl.ds(wid * rows_w, rows_w)], idx_v)

        @pl.loop(0, rows_w)
        def _(j):
            pltpu.sync_copy(ones_v, acc_sh.at[idx_v.at[j]], add=True)

        plsc.subcore_barrier()
        out_off = cid * n_pad + sid * n_slice
        pltpu.sync_copy(acc_sh.at[pl.ds(sid * n_slice, n_slice)],
                        out_hbm.at[pl.ds(out_off, n_slice)])

    return deg_kernel


def _make_layer_kernel(e_pad, n_pad):
    rows_w = e_pad // ROW // NSUB
    ch_rows = 4
    n_chunks = rows_w // ch_rows
    ch_e = ch_rows * ROW
    n_slice = n_pad // NSUB
    mesh = plsc.VectorSubcoreMesh(core_axis_name="c", subcore_axis_name="s")

    idx_t = pltpu.VMEM((ch_rows, ROW), I32)
    msg_t = pltpu.VMEM((ch_e, 16), F32)
    sem_t = pltpu.SemaphoreType.DMA

    @functools.partial(
        pl.kernel,
        mesh=mesh,
        compiler_params=_SC_PARAMS,
        out_type=jax.ShapeDtypeStruct((NCORES * n_pad, 16), F32),
        scratch_types=(
            [idx_t] * 6 + [msg_t] * 3 + [sem_t] * 9
            + [pltpu.VMEM_SHARED((n_pad, 16), F32)]
        ),
    )
    def layer_kernel(hw0_hbm, hw1_hbm, src_hbm, dst_hbm, out_hbm,
                     s0, s1, s2, d0, d1, d2, m0, m1, m2,
                     i0, i1, i2, g0, g1, g2, t0, t1, t2, acc_sh):
        cid = lax.axis_index("c")
        sid = lax.axis_index("s")
        srcs, dsts, msgs = (s0, s1, s2), (d0, d1, d2), (m0, m1, m2)
        isems, gsems, ssems = (i0, i1, i2), (g0, g1, g2), (t0, t1, t2)

        @pl.loop(0, ch_e)
        def _(i):
            m0[i] = jnp.zeros((16,), F32)

        base_n = sid * n_slice

        @pl.loop(0, n_slice // ch_e)
        def _(i):
            pltpu.sync_copy(m0, acc_sh.at[pl.ds(base_n + i * ch_e, ch_e)])

        rem = n_slice % ch_e
        if rem:
            pltpu.sync_copy(m0.at[pl.ds(0, rem)],
                            acc_sh.at[pl.ds(base_n + (n_slice // ch_e) * ch_e,
                                            rem)])
        plsc.subcore_barrier()

        row_base = sid * rows_w

        def fire_idx(c, b):
            rb = row_base + c * ch_rows
            pltpu.async_copy(src_hbm.at[pl.ds(rb, ch_rows)], srcs[b], isems[b])
            pltpu.async_copy(dst_hbm.at[pl.ds(rb, ch_rows)], dsts[b], isems[b])

        def drain_idx(b):
            for _ in range(2):
                pltpu.make_async_copy(src_hbm.at[pl.ds(0, ch_rows)],
                                      srcs[b], isems[b]).wait()

        def fire_gathers(b):
            @pl.when(cid == 0)
            def _():
                for j in range(ch_rows):
                    pltpu.async_copy(hw0_hbm.at[srcs[b].at[j]],
                                     msgs[b].at[pl.ds(j * ROW, ROW)],
                                     gsems[b])

            @pl.when(cid == 1)
            def _():
                for j in range(ch_rows):
                    pltpu.async_copy(hw1_hbm.at[srcs[b].at[j]],
                                     msgs[b].at[pl.ds(j * ROW, ROW)],
                                     gsems[b])

        def drain_gathers(b):
            for j in range(ch_rows):
                pltpu.make_async_copy(
                    hw0_hbm.at[srcs[b].at[j]],
                    msgs[b].at[pl.ds(j * ROW, ROW)], gsems[b]).wait()

        def fire_scatters(b):
            for j in range(ch_rows):
                pltpu.async_copy(msgs[b].at[pl.ds(j * ROW, ROW)],
                                 acc_sh.at[dsts[b].at[j]], ssems[b], add=True)

        def drain_scatters(b):
            for j in range(ch_rows):
                pltpu.make_async_copy(
                    msgs[b].at[pl.ds(j * ROW, ROW)],
                    acc_sh.at[dsts[b].at[j]], ssems[b]).wait()

        def ring_step(c, b, pb, nb, stage):
            drain_idx(b)
            fire_gathers(b)
            if stage >= 1:
                drain_gathers(pb)
                fire_scatters(pb)
            if stage >= 2:
                drain_scatters(nb)
            fire_idx(c + 1, nb)

        fire_idx(0, 0)
        ring_step(0, 0, 2, 1, 0)
        ring_step(1, 1, 0, 2, 1)
        ring_step(2, 2, 1, 0, 2)

        @pl.loop(1, n_chunks // 3)
        def _(g):
            c = g * 3
            ring_step(c, 0, 2, 1, 2)
            ring_step(c + 1, 1, 0, 2, 2)
            ring_step(c + 2, 2, 1, 0, 2)

        drain_gathers(2)
        fire_scatters(2)
        drain_scatters(1)
        drain_scatters(2)
        drain_idx(0)

        plsc.subcore_barrier()
        out_off = cid * n_pad + base_n
        pltpu.sync_copy(acc_sh.at[pl.ds(base_n, n_slice)],
                        out_hbm.at[pl.ds(out_off, n_slice)])

    return layer_kernel


def _tc_prep(deg4, x3, embp, w0, n_pad):
    nb = n_pad // BLK

    def body(deg_ref, x_ref, emb_ref, w0_ref, hw_ref, dis_ref):
        deg = deg_ref[0, 0] + deg_ref[1, 0] + 1.0
        dis = lax.rsqrt(deg)
        dis_ref[0] = dis
        xv = x_ref[0]
        cls = lax.broadcasted_iota(I32, (BLK, 128), 1)
        oh = (xv == cls).astype(F32)
        h0 = jnp.dot(oh, emb_ref[...], preferred_element_type=F32)
        th = dis * h0
        hw1 = jnp.dot(th, w0_ref[...], preferred_element_type=F32)
        hw_ref[0] = hw1[:, :16]
        hw_ref[1] = hw1[:, 16:]

    return pl.pallas_call(
        body,
        grid=(nb,),
        in_specs=[
            pl.BlockSpec((2, 1, BLK, 1), lambda i: (0, i, 0, 0)),
            pl.BlockSpec((1, BLK, 1), lambda i: (i, 0, 0)),
            pl.BlockSpec((128, 32), lambda i: (0, 0)),
            pl.BlockSpec((32, 32), lambda i: (0, 0)),
        ],
        out_specs=[
            pl.BlockSpec((2, BLK, 16), lambda i: (0, i, 0)),
            pl.BlockSpec((1, BLK, 1), lambda i: (i, 0, 0)),
        ],
        out_shape=[
            jax.ShapeDtypeStruct((2, n_pad, 16), F32),
            jax.ShapeDtypeStruct((nb, BLK, 1), F32),
        ],
    )(deg4, x3, embp, w0)


def _tc_layer_packed(accp, hwp, disrep, btile, wkron, n_pad):
    pr = n_pad * 16 // 128
    nbp = pr // 128

    def body(acc_ref, hw_ref, dis_ref, b_ref, wk_ref, out_ref):
        d = dis_ref[...]
        ts = []
        for ci in (0, 1):
            s = acc_ref[ci] + hw_ref[ci]
            h = jnp.maximum(d * s + b_ref[ci], 0.0)
            ts.append(d * h)
        for co in (0, 1):
            out_ref[co] = (
                jnp.dot(ts[0], wk_ref[0, co], preferred_element_type=F32)
                + jnp.dot(ts[1], wk_ref[1, co], preferred_element_type=F32))

    return pl.pallas_call(
        body,
        grid=(nbp,),
        in_specs=[
            pl.BlockSpec((2, 128, 128), lambda i: (0, i, 0)),
            pl.BlockSpec((2, 128, 128), lambda i: (0, i, 0)),
            pl.BlockSpec((128, 128), lambda i: (i, 0)),
            pl.BlockSpec((2, 1, 128), lambda i: (0, 0, 0)),
            pl.BlockSpec((2, 2, 128, 128), lambda i: (0, 0, 0, 0)),
        ],
        out_specs=pl.BlockSpec((2, 128, 128), lambda i: (0, i, 0)),
        out_shape=jax.ShapeDtypeStruct((2, pr, 128), F32),
    )(accp, hwp, disrep, btile, wkron)


def _lstm_step1(b_ih_ref, b_hh_ref):
    gb = b_ih_ref[...] + b_hh_ref[...]
    ii = jax.nn.sigmoid(gb[:, 0:32])
    ff = jax.nn.sigmoid(gb[:, 32:64])
    gg = jnp.tanh(gb[:, 64:96])
    oo = jax.nn.sigmoid(gb[:, 96:128])
    c1 = ii * gg
    h1 = oo * jnp.tanh(c1)
    del ff
    return c1, h1


def _block_h3(acc_ref, hw_ref, dis_ref, b2_ref):
    s = jnp.concatenate(
        [acc_ref[0] + hw_ref[0], acc_ref[1] + hw_ref[1]], axis=1)
    d = dis_ref[0]
    return jnp.maximum(d * s + b2_ref[...], 0.0)


def _block_onehot(batch_ref):
    bv = batch_ref[0]
    seg = lax.broadcasted_iota(I32, (BLK, NSEG), 1)
    return bv == seg


def _seg_sweep_a(h3, oh, q):
    hq = lax.dot_general(h3, q, (((1,), (1,)), ((), ())),
                         preferred_element_type=F32)
    masked = jnp.where(oh, hq, -jnp.inf)
    return jnp.max(masked, axis=0, keepdims=True)


def _seg_sweep_b(h3, oh, q, emax):
    ohf = oh.astype(F32)
    hq = lax.dot_general(h3, q, (((1,), (1,)), ((), ())),
                         preferred_element_type=F32)
    e = jnp.sum(ohf * hq, axis=1, keepdims=True)
    emaxf = jnp.where(jnp.isfinite(emax), emax, 0.0)
    emv = lax.dot_general(ohf, emaxf, (((1,), (1,)), ((), ())),
                          preferred_element_type=F32)
    ee = jnp.exp(e - emv) * jnp.sum(ohf, axis=1, keepdims=True)
    esum = jnp.sum(ohf * ee, axis=0, keepdims=True)
    rnum = lax.dot_general(ohf, ee * h3, (((0,), (0,)), ((), ())),
                           preferred_element_type=F32)
    return esum, rnum


_S2S_SPECS = [
    pl.BlockSpec((2, BLK, 16), lambda i: (0, i, 0)),
    pl.BlockSpec((2, BLK, 16), lambda i: (0, i, 0)),
    pl.BlockSpec((1, BLK, 1), lambda i: (i, 0, 0)),
    pl.BlockSpec((1, BLK, 1), lambda i: (i, 0, 0)),
    pl.BlockSpec((1, 32), lambda i: (0, 0)),
]


def _tc_s2s_a(acc, hwp, dis3, batch3, b2, b_ih, b_hh, n_pad):
    nb = n_pad // BLK

    def body(acc_ref, hw_ref, dis_ref, bat_ref, b2_ref, bih_ref, bhh_ref,
             emax_ref):
        i = pl.program_id(0)

        @pl.when(i == 0)
        def _():
            emax_ref[...] = jnp.full((1, NSEG), -jnp.inf, F32)

        _, h1 = _lstm_step1(bih_ref, bhh_ref)
        q1 = jnp.broadcast_to(h1, (NSEG, 32))
        h3 = _block_h3(acc_ref, hw_ref, dis_ref, b2_ref)
        oh = _block_onehot(bat_ref)
        emax_ref[...] = jnp.maximum(emax_ref[...], _seg_sweep_a(h3, oh, q1))

    return pl.pallas_call(
        body,
        grid=(nb,),
        in_specs=_S2S_SPECS + [
            pl.BlockSpec((1, 128), lambda i: (0, 0)),
            pl.BlockSpec((1, 128), lambda i: (0, 0)),
        ],
        out_specs=pl.BlockSpec((1, NSEG), lambda i: (0, 0)),
        out_shape=jax.ShapeDtypeStruct((1, NSEG), F32),
    )(acc, hwp, dis3, batch3, b2, b_ih, b_hh)


def _tc_s2s_b(acc, hwp, dis3, batch3, b2, b_ih, b_hh, emax1, n_pad):
    nb = n_pad // BLK

    def body(acc_ref, hw_ref, dis_ref, bat_ref, b2_ref, bih_ref, bhh_ref,
             emax_ref, esum_ref, rnum_ref):
        i = pl.program_id(0)

        @pl.when(i == 0)
        def _():
            esum_ref[...] = jnp.zeros((1, NSEG), F32)
            rnum_ref[...] = jnp.zeros((NSEG, 32), F32)

        _, h1 = _lstm_step1(bih_ref, bhh_ref)
        q1 = jnp.broadcast_to(h1, (NSEG, 32))
        h3 = _block_h3(acc_ref, hw_ref, dis_ref, b2_ref)
        oh = _block_onehot(bat_ref)
        esum, rnum = _seg_sweep_b(h3, oh, q1, emax_ref[...])
        esum_ref[...] += esum
        rnum_ref[...] += rnum

    return pl.pallas_call(
        body,
        grid=(nb,),
        in_specs=_S2S_SPECS + [
            pl.BlockSpec((1, 128), lambda i: (0, 0)),
            pl.BlockSpec((1, 128), lambda i: (0, 0)),
            pl.BlockSpec((1, NSEG), lambda i: (0, 0)),
        ],
        out_specs=[
            pl.BlockSpec((1, NSEG), lambda i: (0, 0)),
            pl.BlockSpec((NSEG, 32), lambda i: (0, 0)),
        ],
        out_shape=[
            jax.ShapeDtypeStruct((1, NSEG), F32),
            jax.ShapeDtypeStruct((NSEG, 32), F32),
        ],
    )(acc, hwp, dis3, batch3, b2, b_ih, b_hh, emax1)


def _lstm_step2(bih_ref, bhh_ref, wih_ref, whh_ref, esum1, rnum1):
    c1, h1 = _lstm_step1(bih_ref, bhh_ref)
    q1 = jnp.broadcast_to(h1, (NSEG, 32))
    r1 = rnum1 / (jnp.transpose(esum1) + 1e-16)
    qs1 = jnp.concatenate([q1, r1], axis=1)
    gates = (jnp.dot(qs1, wih_ref[...], preferred_element_type=F32)
             + bih_ref[...]
             + jnp.dot(jnp.broadcast_to(h1, (NSEG, 32)), whh_ref[...],
                       preferred_element_type=F32)
             + bhh_ref[...])
    i2 = jax.nn.sigmoid(gates[:, 0:32])
    f2 = jax.nn.sigmoid(gates[:, 32:64])
    g2 = jnp.tanh(gates[:, 64:96])
    o2 = jax.nn.sigmoid(gates[:, 96:128])
    c2 = f2 * c1 + i2 * g2
    h2 = o2 * jnp.tanh(c2)
    return h2


def _tc_s2s_c(acc, hwp, dis3, batch3, b2, b_ih, b_hh, wihT, whhT,
              esum1, rnum1, n_pad):
    nb = n_pad // BLK

    def body(acc_ref, hw_ref, dis_ref, bat_ref, b2_ref, bih_ref, bhh_ref,
             wih_ref, whh_ref, esum_ref, rnum_ref, emax_ref, q2_ref, q2_s):
        i = pl.program_id(0)

        @pl.when(i == 0)
        def _():
            emax_ref[...] = jnp.full((1, NSEG), -jnp.inf, F32)
            q2_s[...] = _lstm_step2(bih_ref, bhh_ref, wih_ref, whh_ref,
                                    esum_ref[...], rnum_ref[...])
            q2_ref[...] = q2_s[...]

        q2 = q2_s[...]
        h3 = _block_h3(acc_ref, hw_ref, dis_ref, b2_ref)
        oh = _block_onehot(bat_ref)
        emax_ref[...] = jnp.maximum(emax_ref[...], _seg_sweep_a(h3, oh, q2))

    return pl.pallas_call(
        body,
        grid=(nb,),
        in_specs=_S2S_SPECS + [
            pl.BlockSpec((1, 128), lambda i: (0, 0)),
            pl.BlockSpec((1, 128), lambda i: (0, 0)),
            pl.BlockSpec((64, 128), lambda i: (0, 0)),
            pl.BlockSpec((32, 128), lambda i: (0, 0)),
            pl.BlockSpec((1, NSEG), lambda i: (0, 0)),
            pl.BlockSpec((NSEG, 32), lambda i: (0, 0)),
        ],
        out_specs=[
            pl.BlockSpec((1, NSEG), lambda i: (0, 0)),
            pl.BlockSpec((NSEG, 32), lambda i: (0, 0)),
        ],
        out_shape=[
            jax.ShapeDtypeStruct((1, NSEG), F32),
            jax.ShapeDtypeStruct((NSEG, 32), F32),
        ],
        scratch_shapes=[pltpu.VMEM((NSEG, 32), F32)],
    )(acc, hwp, dis3, batch3, b2, b_ih, b_hh, wihT, whhT, esum1, rnum1)


def _tc_s2s_d(acc, hwp, dis3, batch3, b2, emax2, q2, lin0WT, lin0b,
              lin3WT, lin3b, n_pad):
    nb = n_pad // BLK

    def body(acc_ref, hw_ref, dis_ref, bat_ref, b2_ref, emax_ref, q2_ref,
             l0w_ref, l0b_ref, l3w_ref, l3b_ref, out_ref,
             esum_s, rnum_s):
        i = pl.program_id(0)

        @pl.when(i == 0)
        def _():
            esum_s[...] = jnp.zeros((1, NSEG), F32)
            rnum_s[...] = jnp.zeros((NSEG, 32), F32)

        q2 = q2_ref[...]
        h3 = _block_h3(acc_ref, hw_ref, dis_ref, b2_ref)
        oh = _block_onehot(bat_ref)
        esum, rnum = _seg_sweep_b(h3, oh, q2, emax_ref[...])
        esum_s[...] += esum
        rnum_s[...] += rnum

        @pl.when(i == nb - 1)
        def _():
            r2 = rnum_s[...] / (jnp.transpose(esum_s[...]) + 1e-16)
            qs2 = jnp.concatenate([q2, r2], axis=1)
            z = jnp.maximum(
                jnp.dot(qs2, l0w_ref[...], preferred_element_type=F32)
                + l0b_ref[...], 0.0)
            out_ref[...] = (jnp.dot(z, l3w_ref[...],
                                    preferred_element_type=F32)
                            + l3b_ref[...])

    return pl.pallas_call(
        body,
        grid=(nb,),
        in_specs=_S2S_SPECS + [
            pl.BlockSpec((1, NSEG), lambda i: (0, 0)),
            pl.BlockSpec((NSEG, 32), lambda i: (0, 0)),
            pl.BlockSpec((64, 32), lambda i: (0, 0)),
            pl.BlockSpec((1, 32), lambda i: (0, 0)),
            pl.BlockSpec((32, 1), lambda i: (0, 0)),
            pl.BlockSpec((1, 1), lambda i: (0, 0)),
        ],
        out_specs=pl.BlockSpec((NSEG, 1), lambda i: (0, 0)),
        out_shape=jax.ShapeDtypeStruct((NSEG, 1), F32),
        scratch_shapes=[
            pltpu.VMEM((1, NSEG), F32),
            pltpu.VMEM((NSEG, 32), F32),
        ],
    )(acc, hwp, dis3, batch3, b2, emax2, q2, lin0WT, lin0b, lin3WT, lin3b)


def kernel(x, edge_index, batch, emb, W0, b0, W1, b1, W2, b2,
           W_ih, W_hh, b_ih, b_hh, lin0_W, lin0_b, lin3_W, lin3_b):
    n = x.shape[0]
    e = edge_index.shape[1]

    n_pad = _ceil_to(n, NSUB * ROW)
    if n_pad == n:
        n_pad += NSUB * ROW
    pad_rows = n_pad - n
    e_pad = _ceil_to(e, NSUB * ROW * 12)
    nb = n_pad // BLK

    src = edge_index[0].astype(I32)
    dst = edge_index[1].astype(I32)
    pe = e_pad + 4 * ROW - e
    pad_ar = jnp.arange(pe, dtype=I32)
    src_p = jnp.concatenate([src, (pad_ar * 97) % n])
    dst_p = jnp.concatenate([dst, n + pad_ar % pad_rows])
    src2d = src_p.reshape(e_pad // ROW + 4, ROW)
    dst2d = dst_p.reshape(e_pad // ROW + 4, ROW)

    x_p = jnp.concatenate([x.astype(I32), jnp.zeros((n_pad - n,), I32)])
    x3 = x_p.reshape(nb, BLK, 1)
    batch_p = jnp.concatenate(
        [batch.astype(I32), jnp.full((n_pad - n,), NSEG, I32)])
    batch3 = batch_p.reshape(nb, BLK, 1)

    embp = jnp.zeros((128, 32), F32).at[:emb.shape[0]].set(emb)
    b0r = b0.reshape(1, 32)
    b1r = b1.reshape(1, 32)
    b2r = b2.reshape(1, 32)
    bihr = b_ih.reshape(1, 128)
    bhhr = b_hh.reshape(1, 128)
    wihT = W_ih.T
    whhT = W_hh.T
    lin0WT = lin0_W.T
    lin0br = lin0_b.reshape(1, 32)
    lin3WT = lin3_W.T
    lin3br = lin3_b.reshape(1, 1)

    deg_flat = _make_deg_kernel(e_pad, n_pad)(dst2d)
    deg4 = deg_flat.reshape(2, nb, BLK, 1)

    hw1, dis3 = _tc_prep(deg4, x3, embp, W0, n_pad)

    layer = _make_layer_kernel(e_pad, n_pad)

    pr = n_pad * 16 // 128
    disrep = jnp.repeat(dis3.reshape(n_pad), 16).reshape(pr, 128)
    eye8 = jnp.eye(8, dtype=F32)

    def kron_w(w):
        return jnp.stack([
            jnp.stack([jnp.kron(eye8, w[16 * ci:16 * ci + 16,
                                        16 * co:16 * co + 16])
                       for co in (0, 1)])
            for ci in (0, 1)])

    def btile_of(b):
        return jnp.stack([jnp.tile(b[:16], 8),
                          jnp.tile(b[16:], 8)]).reshape(2, 1, 128)

    hw1p = hw1.reshape(2, pr, 128)
    acc1p = layer(hw1[0], hw1[1], src2d, dst2d).reshape(2, pr, 128)
    hw2p = _tc_layer_packed(acc1p, hw1p, disrep, btile_of(b0),
                            kron_w(W1), n_pad)
    hw2v = hw2p.reshape(2, n_pad, 16)
    acc2p = layer(hw2v[0], hw2v[1], src2d, dst2d).reshape(2, pr, 128)
    hw3p = _tc_layer_packed(acc2p, hw2p, disrep, btile_of(b1),
                            kron_w(W2), n_pad)
    hw3 = hw3p.reshape(2, n_pad, 16)
    acc3 = layer(hw3[0], hw3[1], src2d, dst2d).reshape(2, n_pad, 16)

    emax1 = _tc_s2s_a(acc3, hw3, dis3, batch3, b2r, bihr, bhhr, n_pad)
    esum1, rnum1 = _tc_s2s_b(acc3, hw3, dis3, batch3, b2r, bihr, bhhr,
                             emax1, n_pad)
    emax2, q2 = _tc_s2s_c(acc3, hw3, dis3, batch3, b2r, bihr, bhhr,
                          wihT, whhT, esum1, rnum1, n_pad)
    out = _tc_s2s_d(acc3, hw3, dis3, batch3, b2r, emax2, q2,
                    lin0WT, lin0br, lin3WT, lin3br, n_pad)
    return out


# speedup vs baseline: 25.1542x; 1.0332x over previous
import functools

import jax
import jax.numpy as jnp
from jax import lax
from jax.experimental import pallas as pl
from jax.experimental.pallas import tpu as pltpu
from jax.experimental.pallas import tpu_sc as plsc

F32 = jnp.float32
I32 = jnp.int32

NCORES = 2
NSUB = 16
ROW = 128
BLK = 1024
NSEG = 256

_SC_PARAMS = pltpu.CompilerParams(use_tc_tiling_on_sc=False)


def _ceil_to(a, m):
    return (a + m - 1) // m * m


def _make_deg_kernel(e_pad, n_pad):
    rows_w = e_pad // ROW // (NCORES * NSUB)
    n_slice = n_pad // NSUB
    mesh = plsc.VectorSubcoreMesh(core_axis_name="c", subcore_axis_name="s")

    @functools.partial(
        pl.kernel,
        mesh=mesh,
        compiler_params=_SC_PARAMS,
        out_type=jax.ShapeDtypeStruct((NCORES * n_pad,), F32),
        scratch_types=[
            pltpu.VMEM((rows_w, ROW), I32),
            pltpu.VMEM((n_slice,), F32),
            pltpu.VMEM((ROW,), F32),
            pltpu.VMEM_SHARED((n_pad,), F32),
        ],
    )
    def deg_kernel(dst_hbm, out_hbm, idx_v, zero_v, ones_v, acc_sh):
        cid = lax.axis_index("c")
        sid = lax.axis_index("s")

        @pl.loop(0, n_slice // 16)
        def _(i):
            zero_v[pl.ds(i * 16, 16)] = jnp.zeros((16,), F32)

        @pl.loop(0, ROW // 16)
        def _(i):
            ones_v[pl.ds(i * 16, 16)] = jnp.ones((16,), F32)

        pltpu.sync_copy(zero_v, acc_sh.at[pl.ds(sid * n_slice, n_slice)])
        plsc.subcore_barrier()

        wid = cid * NSUB + sid
        pltpu.sync_copy(dst_hbm.at[pl.ds(wid * rows_w, rows_w)], idx_v)

        @pl.loop(0, rows_w)
        def _(j):
            pltpu.sync_copy(ones_v, acc_sh.at[idx_v.at[j]], add=True)

        plsc.subcore_barrier()
        out_off = cid * n_pad + sid * n_slice
        pltpu.sync_copy(acc_sh.at[pl.ds(sid * n_slice, n_slice)],
                        out_hbm.at[pl.ds(out_off, n_slice)])

    return deg_kernel


def _make_layer_kernel(e_pad, n_pad):
    rows_w = e_pad // ROW // NSUB
    ch_rows = 4
    n_chunks = rows_w // ch_rows
    ch_e = ch_rows * ROW
    n_slice = n_pad // NSUB
    mesh = plsc.VectorSubcoreMesh(core_axis_name="c", subcore_axis_name="s")

    idx_t = pltpu.VMEM((ch_rows, ROW), I32)
    msg_t = pltpu.VMEM((ch_e, 16), F32)
    sem_t = pltpu.SemaphoreType.DMA

    @functools.partial(
        pl.kernel,
        mesh=mesh,
        compiler_params=_SC_PARAMS,
        out_type=jax.ShapeDtypeStruct((NCORES * n_pad, 16), F32),
        scratch_types=(
            [idx_t] * 6 + [msg_t] * 3 + [sem_t] * 9
            + [pltpu.VMEM_SHARED((n_pad, 16), F32)]
        ),
    )
    def layer_kernel(hw0_hbm, hw1_hbm, src_hbm, dst_hbm, out_hbm,
                     s0, s1, s2, d0, d1, d2, m0, m1, m2,
                     i0, i1, i2, g0, g1, g2, t0, t1, t2, acc_sh):
        cid = lax.axis_index("c")
        sid = lax.axis_index("s")
        srcs, dsts, msgs = (s0, s1, s2), (d0, d1, d2), (m0, m1, m2)
        isems, gsems, ssems = (i0, i1, i2), (g0, g1, g2), (t0, t1, t2)

        @pl.loop(0, ch_e)
        def _(i):
            m0[i] = jnp.zeros((16,), F32)

        base_n = sid * n_slice

        @pl.loop(0, n_slice // ch_e)
        def _(i):
            pltpu.sync_copy(m0, acc_sh.at[pl.ds(base_n + i * ch_e, ch_e)])

        rem = n_slice % ch_e
        if rem:
            pltpu.sync_copy(m0.at[pl.ds(0, rem)],
                            acc_sh.at[pl.ds(base_n + (n_slice // ch_e) * ch_e,
                                            rem)])
        plsc.subcore_barrier()

        row_base = sid * rows_w

        def fire_idx(c, b):
            rb = row_base + c * ch_rows
            pltpu.async_copy(src_hbm.at[pl.ds(rb, ch_rows)], srcs[b], isems[b])
            pltpu.async_copy(dst_hbm.at[pl.ds(rb, ch_rows)], dsts[b], isems[b])

        def drain_idx(b):
            for _ in range(2):
                pltpu.make_async_copy(src_hbm.at[pl.ds(0, ch_rows)],
                                      srcs[b], isems[b]).wait()

        def fire_gathers(b):
            @pl.when(cid == 0)
            def _():
                for j in range(ch_rows):
                    pltpu.async_copy(hw0_hbm.at[srcs[b].at[j]],
                                     msgs[b].at[pl.ds(j * ROW, ROW)],
                                     gsems[b])

            @pl.when(cid == 1)
            def _():
                for j in range(ch_rows):
                    pltpu.async_copy(hw1_hbm.at[srcs[b].at[j]],
                                     msgs[b].at[pl.ds(j * ROW, ROW)],
                                     gsems[b])

        def drain_gathers(b):
            for j in range(ch_rows):
                pltpu.make_async_copy(
                    hw0_hbm.at[srcs[b].at[j]],
                    msgs[b].at[pl.ds(j * ROW, ROW)], gsems[b]).wait()

        def fire_scatters(b):
            for j in range(ch_rows):
                pltpu.async_copy(msgs[b].at[pl.ds(j * ROW, ROW)],
                                 acc_sh.at[dsts[b].at[j]], ssems[b], add=True)

        def drain_scatters(b):
            for j in range(ch_rows):
                pltpu.make_async_copy(
                    msgs[b].at[pl.ds(j * ROW, ROW)],
                    acc_sh.at[dsts[b].at[j]], ssems[b]).wait()

        def ring_step(c, b, pb, nb, stage):
            drain_idx(b)
            fire_gathers(b)
            if stage >= 1:
                drain_gathers(pb)
                fire_scatters(pb)
            if stage >= 2:
                drain_scatters(nb)
            fire_idx(c + 1, nb)

        fire_idx(0, 0)
        ring_step(0, 0, 2, 1, 0)
        ring_step(1, 1, 0, 2, 1)
        ring_step(2, 2, 1, 0, 2)

        @pl.loop(1, n_chunks // 3)
        def _(g):
            c = g * 3
            ring_step(c, 0, 2, 1, 2)
            ring_step(c + 1, 1, 0, 2, 2)
            ring_step(c + 2, 2, 1, 0, 2)

        drain_gathers(2)
        fire_scatters(2)
        drain_scatters(1)
        drain_scatters(2)
        drain_idx(0)

        plsc.subcore_barrier()
        out_off = cid * n_pad + base_n
        pltpu.sync_copy(acc_sh.at[pl.ds(base_n, n_slice)],
                        out_hbm.at[pl.ds(out_off, n_slice)])

    return layer_kernel


def _tc_prep(deg4, x3, embp, w0, n_pad):
    nb = n_pad // BLK

    def body(deg_ref, x_ref, emb_ref, w0_ref, hw_ref, dis_ref):
        deg = deg_ref[0, 0][:, :1] + deg_ref[1, 0][:, :1] + 1.0
        dis = lax.rsqrt(deg)
        dis_ref[0] = jnp.broadcast_to(dis, (BLK, 8))
        xv = x_ref[0][:, :1]
        cls = lax.broadcasted_iota(I32, (BLK, 128), 1)
        oh = (xv == cls).astype(F32)
        h0 = jnp.dot(oh, emb_ref[...], preferred_element_type=F32)
        th = dis * h0
        hw1 = jnp.dot(th, w0_ref[...], preferred_element_type=F32)
        hw_ref[0] = hw1[:, :16]
        hw_ref[1] = hw1[:, 16:]

    return pl.pallas_call(
        body,
        grid=(nb,),
        in_specs=[
            pl.BlockSpec((2, 1, BLK, 8), lambda i: (0, i, 0, 0)),
            pl.BlockSpec((1, BLK, 8), lambda i: (i, 0, 0)),
            pl.BlockSpec((128, 32), lambda i: (0, 0)),
            pl.BlockSpec((32, 32), lambda i: (0, 0)),
        ],
        out_specs=[
            pl.BlockSpec((2, BLK, 16), lambda i: (0, i, 0)),
            pl.BlockSpec((1, BLK, 8), lambda i: (i, 0, 0)),
        ],
        out_shape=[
            jax.ShapeDtypeStruct((2, n_pad, 16), F32),
            jax.ShapeDtypeStruct((nb, BLK, 8), F32),
        ],
    )(deg4, x3, embp, w0)


def _tc_layer_packed(accp, hwp, disrep, btile, wkron, n_pad):
    pr = n_pad * 16 // 128
    nbp = pr // 128

    def body(acc_ref, hw_ref, dis_ref, b_ref, wk_ref, out_ref):
        d = dis_ref[...]
        ts = []
        for ci in (0, 1):
            s = acc_ref[ci] + hw_ref[ci]
            h = jnp.maximum(d * s + b_ref[ci], 0.0)
            ts.append(d * h)
        for co in (0, 1):
            out_ref[co] = (
                jnp.dot(ts[0], wk_ref[0, co], preferred_element_type=F32)
                + jnp.dot(ts[1], wk_ref[1, co], preferred_element_type=F32))

    return pl.pallas_call(
        body,
        grid=(nbp,),
        in_specs=[
            pl.BlockSpec((2, 128, 128), lambda i: (0, i, 0)),
            pl.BlockSpec((2, 128, 128), lambda i: (0, i, 0)),
            pl.BlockSpec((128, 128), lambda i: (i, 0)),
            pl.BlockSpec((2, 1, 128), lambda i: (0, 0, 0)),
            pl.BlockSpec((2, 2, 128, 128), lambda i: (0, 0, 0, 0)),
        ],
        out_specs=pl.BlockSpec((2, 128, 128), lambda i: (0, i, 0)),
        out_shape=jax.ShapeDtypeStruct((2, pr, 128), F32),
    )(accp, hwp, disrep, btile, wkron)


def _lstm_step1(b_ih_ref, b_hh_ref):
    gb = b_ih_ref[...] + b_hh_ref[...]
    ii = jax.nn.sigmoid(gb[:, 0:32])
    ff = jax.nn.sigmoid(gb[:, 32:64])
    gg = jnp.tanh(gb[:, 64:96])
    oo = jax.nn.sigmoid(gb[:, 96:128])
    c1 = ii * gg
    h1 = oo * jnp.tanh(c1)
    del ff
    return c1, h1


def _block_h3(acc_ref, hw_ref, dis_ref, b2_ref):
    s = jnp.concatenate(
        [acc_ref[0] + hw_ref[0], acc_ref[1] + hw_ref[1]], axis=1)
    d = dis_ref[0][:, :1]
    return jnp.maximum(d * s + b2_ref[...], 0.0)


def _block_onehot(batch_ref):
    bv = batch_ref[0][:, :1]
    seg = lax.broadcasted_iota(I32, (BLK, NSEG), 1)
    return bv == seg


def _seg_sweep_a(h3, oh, q):
    hq = lax.dot_general(h3, q, (((1,), (1,)), ((), ())),
                         preferred_element_type=F32)
    masked = jnp.where(oh, hq, -jnp.inf)
    return jnp.max(masked, axis=0, keepdims=True)


def _seg_sweep_b(h3, oh, q, emax):
    hq = lax.dot_general(h3, q, (((1,), (1,)), ((), ())),
                         preferred_element_type=F32)
    emaxf = jnp.where(jnp.isfinite(emax), emax, 0.0)
    z = jnp.exp(jnp.where(oh, hq - emaxf, -100.0))
    esum = jnp.sum(z, axis=0, keepdims=True)
    rnum = lax.dot_general(z, h3, (((0,), (0,)), ((), ())),
                           preferred_element_type=F32)
    return esum, rnum


_S2S_SPECS = [
    pl.BlockSpec((2, BLK, 16), lambda i: (0, i, 0)),
    pl.BlockSpec((2, BLK, 16), lambda i: (0, i, 0)),
    pl.BlockSpec((1, BLK, 8), lambda i: (i, 0, 0)),
    pl.BlockSpec((1, BLK, 8), lambda i: (i, 0, 0)),
    pl.BlockSpec((1, 32), lambda i: (0, 0)),
]


def _tc_s2s_a(acc, hwp, dis3, batch3, b2, b_ih, b_hh, n_pad):
    nb = n_pad // BLK

    def body(acc_ref, hw_ref, dis_ref, bat_ref, b2_ref, bih_ref, bhh_ref,
             emax_ref):
        i = pl.program_id(0)

        @pl.when(i == 0)
        def _():
            emax_ref[...] = jnp.full((1, NSEG), -jnp.inf, F32)

        _, h1 = _lstm_step1(bih_ref, bhh_ref)
        q1 = jnp.broadcast_to(h1, (NSEG, 32))
        h3 = _block_h3(acc_ref, hw_ref, dis_ref, b2_ref)
        oh = _block_onehot(bat_ref)
        emax_ref[...] = jnp.maximum(emax_ref[...], _seg_sweep_a(h3, oh, q1))

    return pl.pallas_call(
        body,
        grid=(nb,),
        in_specs=_S2S_SPECS + [
            pl.BlockSpec((1, 128), lambda i: (0, 0)),
            pl.BlockSpec((1, 128), lambda i: (0, 0)),
        ],
        out_specs=pl.BlockSpec((1, NSEG), lambda i: (0, 0)),
        out_shape=jax.ShapeDtypeStruct((1, NSEG), F32),
    )(acc, hwp, dis3, batch3, b2, b_ih, b_hh)


def _tc_s2s_b(acc, hwp, dis3, batch3, b2, b_ih, b_hh, emax1, n_pad):
    nb = n_pad // BLK

    def body(acc_ref, hw_ref, dis_ref, bat_ref, b2_ref, bih_ref, bhh_ref,
             emax_ref, esum_ref, rnum_ref):
        i = pl.program_id(0)

        @pl.when(i == 0)
        def _():
            esum_ref[...] = jnp.zeros((1, NSEG), F32)
            rnum_ref[...] = jnp.zeros((NSEG, 32), F32)

        _, h1 = _lstm_step1(bih_ref, bhh_ref)
        q1 = jnp.broadcast_to(h1, (NSEG, 32))
        h3 = _block_h3(acc_ref, hw_ref, dis_ref, b2_ref)
        oh = _block_onehot(bat_ref)
        esum, rnum = _seg_sweep_b(h3, oh, q1, emax_ref[...])
        esum_ref[...] += esum
        rnum_ref[...] += rnum

    return pl.pallas_call(
        body,
        grid=(nb,),
        in_specs=_S2S_SPECS + [
            pl.BlockSpec((1, 128), lambda i: (0, 0)),
            pl.BlockSpec((1, 128), lambda i: (0, 0)),
            pl.BlockSpec((1, NSEG), lambda i: (0, 0)),
        ],
        out_specs=[
            pl.BlockSpec((1, NSEG), lambda i: (0, 0)),
            pl.BlockSpec((NSEG, 32), lambda i: (0, 0)),
        ],
        out_shape=[
            jax.ShapeDtypeStruct((1, NSEG), F32),
            jax.ShapeDtypeStruct((NSEG, 32), F32),
        ],
    )(acc, hwp, dis3, batch3, b2, b_ih, b_hh, emax1)


def _lstm_step2(bih_ref, bhh_ref, wih_ref, whh_ref, esum1, rnum1):
    c1, h1 = _lstm_step1(bih_ref, bhh_ref)
    q1 = jnp.broadcast_to(h1, (NSEG, 32))
    r1 = rnum1 / (jnp.transpose(esum1) + 1e-16)
    qs1 = jnp.concatenate([q1, r1], axis=1)
    gates = (jnp.dot(qs1, wih_ref[...], preferred_element_type=F32)
             + bih_ref[...]
             + jnp.dot(jnp.broadcast_to(h1, (NSEG, 32)), whh_ref[...],
                       preferred_element_type=F32)
             + bhh_ref[...])
    i2 = jax.nn.sigmoid(gates[:, 0:32])
    f2 = jax.nn.sigmoid(gates[:, 32:64])
    g2 = jnp.tanh(gates[:, 64:96])
    o2 = jax.nn.sigmoid(gates[:, 96:128])
    c2 = f2 * c1 + i2 * g2
    h2 = o2 * jnp.tanh(c2)
    return h2


def _tc_s2s_c(acc, hwp, dis3, batch3, b2, b_ih, b_hh, wihT, whhT,
              esum1, rnum1, n_pad):
    nb = n_pad // BLK

    def body(acc_ref, hw_ref, dis_ref, bat_ref, b2_ref, bih_ref, bhh_ref,
             wih_ref, whh_ref, esum_ref, rnum_ref, emax_ref, q2_ref, q2_s):
        i = pl.program_id(0)

        @pl.when(i == 0)
        def _():
            emax_ref[...] = jnp.full((1, NSEG), -jnp.inf, F32)
            q2_s[...] = _lstm_step2(bih_ref, bhh_ref, wih_ref, whh_ref,
                                    esum_ref[...], rnum_ref[...])
            q2_ref[...] = q2_s[...]

        q2 = q2_s[...]
        h3 = _block_h3(acc_ref, hw_ref, dis_ref, b2_ref)
        oh = _block_onehot(bat_ref)
        emax_ref[...] = jnp.maximum(emax_ref[...], _seg_sweep_a(h3, oh, q2))

    return pl.pallas_call(
        body,
        grid=(nb,),
        in_specs=_S2S_SPECS + [
            pl.BlockSpec((1, 128), lambda i: (0, 0)),
            pl.BlockSpec((1, 128), lambda i: (0, 0)),
            pl.BlockSpec((64, 128), lambda i: (0, 0)),
            pl.BlockSpec((32, 128), lambda i: (0, 0)),
            pl.BlockSpec((1, NSEG), lambda i: (0, 0)),
            pl.BlockSpec((NSEG, 32), lambda i: (0, 0)),
        ],
        out_specs=[
            pl.BlockSpec((1, NSEG), lambda i: (0, 0)),
            pl.BlockSpec((NSEG, 32), lambda i: (0, 0)),
        ],
        out_shape=[
            jax.ShapeDtypeStruct((1, NSEG), F32),
            jax.ShapeDtypeStruct((NSEG, 32), F32),
        ],
        scratch_shapes=[pltpu.VMEM((NSEG, 32), F32)],
    )(acc, hwp, dis3, batch3, b2, b_ih, b_hh, wihT, whhT, esum1, rnum1)


def _tc_s2s_d(acc, hwp, dis3, batch3, b2, emax2, q2, lin0WT, lin0b,
              lin3WT, lin3b, n_pad):
    nb = n_pad // BLK

    def body(acc_ref, hw_ref, dis_ref, bat_ref, b2_ref, emax_ref, q2_ref,
             l0w_ref, l0b_ref, l3w_ref, l3b_ref, out_ref,
             esum_s, rnum_s):
        i = pl.program_id(0)

        @pl.when(i == 0)
        def _():
            esum_s[...] = jnp.zeros((1, NSEG), F32)
            rnum_s[...] = jnp.zeros((NSEG, 32), F32)

        q2 = q2_ref[...]
        h3 = _block_h3(acc_ref, hw_ref, dis_ref, b2_ref)
        oh = _block_onehot(bat_ref)
        esum, rnum = _seg_sweep_b(h3, oh, q2, emax_ref[...])
        esum_s[...] += esum
        rnum_s[...] += rnum

        @pl.when(i == nb - 1)
        def _():
            r2 = rnum_s[...] / (jnp.transpose(esum_s[...]) + 1e-16)
            qs2 = jnp.concatenate([q2, r2], axis=1)
            z = jnp.maximum(
                jnp.dot(qs2, l0w_ref[...], preferred_element_type=F32)
                + l0b_ref[...], 0.0)
            out_ref[...] = (jnp.dot(z, l3w_ref[...],
                                    preferred_element_type=F32)
                            + l3b_ref[...])

    return pl.pallas_call(
        body,
        grid=(nb,),
        in_specs=_S2S_SPECS + [
            pl.BlockSpec((1, NSEG), lambda i: (0, 0)),
            pl.BlockSpec((NSEG, 32), lambda i: (0, 0)),
            pl.BlockSpec((64, 32), lambda i: (0, 0)),
            pl.BlockSpec((1, 32), lambda i: (0, 0)),
            pl.BlockSpec((32, 1), lambda i: (0, 0)),
            pl.BlockSpec((1, 1), lambda i: (0, 0)),
        ],
        out_specs=pl.BlockSpec((NSEG, 1), lambda i: (0, 0)),
        out_shape=jax.ShapeDtypeStruct((NSEG, 1), F32),
        scratch_shapes=[
            pltpu.VMEM((1, NSEG), F32),
            pltpu.VMEM((NSEG, 32), F32),
        ],
    )(acc, hwp, dis3, batch3, b2, emax2, q2, lin0WT, lin0b, lin3WT, lin3b)


def kernel(x, edge_index, batch, emb, W0, b0, W1, b1, W2, b2,
           W_ih, W_hh, b_ih, b_hh, lin0_W, lin0_b, lin3_W, lin3_b):
    n = x.shape[0]
    e = edge_index.shape[1]

    n_pad = _ceil_to(n, NSUB * ROW)
    if n_pad == n:
        n_pad += NSUB * ROW
    pad_rows = n_pad - n
    e_pad = _ceil_to(e, NSUB * ROW * 12)
    nb = n_pad // BLK

    src = edge_index[0].astype(I32)
    dst = edge_index[1].astype(I32)
    pe = e_pad + 4 * ROW - e
    pad_ar = jnp.arange(pe, dtype=I32)
    src_p = jnp.concatenate([src, (pad_ar * 97) % n])
    dst_p = jnp.concatenate([dst, n + pad_ar % pad_rows])
    src2d = src_p.reshape(e_pad // ROW + 4, ROW)
    dst2d = dst_p.reshape(e_pad // ROW + 4, ROW)

    x_p = jnp.concatenate([x.astype(I32), jnp.zeros((n_pad - n,), I32)])
    x3 = jnp.broadcast_to(x_p[:, None], (n_pad, 8)).reshape(nb, BLK, 8)
    batch_p = jnp.concatenate(
        [batch.astype(I32), jnp.full((n_pad - n,), NSEG, I32)])
    batch3 = jnp.broadcast_to(batch_p[:, None], (n_pad, 8)).reshape(
        nb, BLK, 8)

    embp = jnp.zeros((128, 32), F32).at[:emb.shape[0]].set(emb)
    b0r = b0.reshape(1, 32)
    b1r = b1.reshape(1, 32)
    b2r = b2.reshape(1, 32)
    bihr = b_ih.reshape(1, 128)
    bhhr = b_hh.reshape(1, 128)
    wihT = W_ih.T
    whhT = W_hh.T
    lin0WT = lin0_W.T
    lin0br = lin0_b.reshape(1, 32)
    lin3WT = lin3_W.T
    lin3br = lin3_b.reshape(1, 1)

    deg_flat = _make_deg_kernel(e_pad, n_pad)(dst2d)
    deg4 = jnp.broadcast_to(
        deg_flat.reshape(2, n_pad, 1), (2, n_pad, 8)).reshape(2, nb, BLK, 8)

    hw1, dis3 = _tc_prep(deg4, x3, embp, W0, n_pad)

    layer = _make_layer_kernel(e_pad, n_pad)

    pr = n_pad * 16 // 128
    disrep = jnp.repeat(dis3[:, :, 0].reshape(n_pad), 16).reshape(pr, 128)
    eye8 = jnp.eye(8, dtype=F32)

    def kron_w(w):
        return jnp.stack([
            jnp.stack([jnp.kron(eye8, w[16 * ci:16 * ci + 16,
                                        16 * co:16 * co + 16])
                       for co in (0, 1)])
            for ci in (0, 1)])

    def btile_of(b):
        return jnp.stack([jnp.tile(b[:16], 8),
                          jnp.tile(b[16:], 8)]).reshape(2, 1, 128)

    hw1p = hw1.reshape(2, pr, 128)
    acc1p = layer(hw1[0], hw1[1], src2d, dst2d).reshape(2, pr, 128)
    hw2p = _tc_layer_packed(acc1p, hw1p, disrep, btile_of(b0),
                            kron_w(W1), n_pad)
    hw2v = hw2p.reshape(2, n_pad, 16)
    acc2p = layer(hw2v[0], hw2v[1], src2d, dst2d).reshape(2, pr, 128)
    hw3p = _tc_layer_packed(acc2p, hw2p, disrep, btile_of(b1),
                            kron_w(W2), n_pad)
    hw3 = hw3p.reshape(2, n_pad, 16)
    acc3 = layer(hw3[0], hw3[1], src2d, dst2d).reshape(2, n_pad, 16)

    emax1 = _tc_s2s_a(acc3, hw3, dis3, batch3, b2r, bihr, bhhr, n_pad)
    esum1, rnum1 = _tc_s2s_b(acc3, hw3, dis3, batch3, b2r, bihr, bhhr,
                             emax1, n_pad)
    emax2, q2 = _tc_s2s_c(acc3, hw3, dis3, batch3, b2r, bihr, bhhr,
                          wihT, whhT, esum1, rnum1, n_pad)
    out = _tc_s2s_d(acc3, hw3, dis3, batch3, b2r, emax2, q2,
                    lin0WT, lin0br, lin3WT, lin3br, n_pad)
    return out
